# Initial kernel scaffold; baseline (speedup 1.0000x reference)
import jax
import jax.numpy as jnp
from jax.experimental import pallas as pl


def kernel(cell_x, well_x, c2c_edge_attr, params, c2c_edge_index, c2w_src, c2w_dst):
    raise NotImplementedError("write your pallas kernel here")


# speedup vs baseline: 4.9469x; 4.9469x over previous
import functools

import jax
import jax.numpy as jnp
from jax import lax
from jax.experimental import pallas as pl
from jax.experimental.pallas import tpu as pltpu
from jax.experimental.pallas import tpu_sc as plsc

F32 = jnp.float32
I32 = jnp.int32

NCELL = 50000
NPAD = 50176
NBLK = 98
BLK = 512
NWELL = 500
NWPAD = 512
ECC = 800000
ECC_PAD = 16 * 50176
ECW = 32000
ECW_PAD = 32768
H = 64


def _tc_h0_body(x_ref, wpre_ref, bpre_ref, h0_ref):
    h = jnp.dot(x_ref[...], wpre_ref[...], preferred_element_type=F32)
    h = h + bpre_ref[...]
    for q in range(4):
        h0_ref[q, :, :] = h[:, 16 * q:16 * q + 16]


def _tc_embed_body(x_ref, flag_ref, bprev_ref, w_ref, as_ref, ad_ref,
                   hs_ref, ss_ref, sd_ref, mxs_ref, mxd_ref):
    g = pl.program_id(0)
    x = jnp.concatenate([x_ref[0], x_ref[1], x_ref[2], x_ref[3]], axis=1)
    f = flag_ref[0, 0]
    h = jnp.where(f > 0.0, jnp.maximum(x + bprev_ref[...], 0.0), x)
    _tc_tail(g, h, w_ref, as_ref, ad_ref, hs_ref, ss_ref, sd_ref,
             mxs_ref, mxd_ref)


def _tc_tail(g, h, w_ref, as_ref, ad_ref, hs_ref, ss_ref, sd_ref,
             mxs_ref, mxd_ref):
    hs = jnp.dot(h, w_ref[...], preferred_element_type=F32)
    for q in range(4):
        hs_ref[q, :, :] = hs[:, 16 * q:16 * q + 16]
    ss_t = lax.dot_general(as_ref[...], hs, (((1,), (1,)), ((), ())),
                           preferred_element_type=F32)
    sd_t = lax.dot_general(ad_ref[...], hs, (((1,), (1,)), ((), ())),
                           preferred_element_type=F32)
    ss_ref[...] = ss_t.reshape(1, 1, BLK)
    sd_ref[...] = sd_t.reshape(1, 1, BLK)
    rid = g * BLK + lax.broadcasted_iota(I32, (1, BLK), 1)
    valid = rid < NCELL
    ssm = jnp.max(jnp.where(valid, ss_t, -1e30))
    sdm = jnp.max(jnp.where(valid, sd_t, -1e30))

    @pl.when(g == 0)
    def _():
        mxs_ref[...] = jnp.full((1, 128), -1e30, F32)
        mxd_ref[...] = jnp.full((1, 128), -1e30, F32)

    mxs_ref[...] = jnp.maximum(mxs_ref[...], ssm)
    mxd_ref[...] = jnp.maximum(mxd_ref[...], sdm)


def _embed_outs():
    return (
        jax.ShapeDtypeStruct((4, NPAD, 16), F32),
        jax.ShapeDtypeStruct((NBLK, 1, BLK), F32),
        jax.ShapeDtypeStruct((NBLK, 1, BLK), F32),
        jax.ShapeDtypeStruct((1, 128), F32),
        jax.ShapeDtypeStruct((1, 128), F32),
    )


def _embed_out_specs():
    return (
        pl.BlockSpec((4, BLK, 16), lambda g: (0, g, 0)),
        pl.BlockSpec((1, 1, BLK), lambda g: (g, 0, 0)),
        pl.BlockSpec((1, 1, BLK), lambda g: (g, 0, 0)),
        pl.BlockSpec((1, 128), lambda g: (0, 0)),
        pl.BlockSpec((1, 128), lambda g: (0, 0)),
    )


def _full(shape):
    return pl.BlockSpec(shape, lambda g: tuple(0 for _ in shape))


_tc_h0 = pl.pallas_call(
    _tc_h0_body,
    grid=(NBLK,),
    in_specs=[
        pl.BlockSpec((BLK, 128), lambda g: (g, 0)),
        _full((128, H)), _full((1, H)),
    ],
    out_specs=pl.BlockSpec((4, BLK, 16), lambda g: (0, g, 0)),
    out_shape=jax.ShapeDtypeStruct((4, NPAD, 16), F32),
)

_tc_embed = pl.pallas_call(
    _tc_embed_body,
    grid=(NBLK,),
    in_specs=[
        pl.BlockSpec((4, BLK, 16), lambda g: (0, g, 0)),
        _full((1, 128)),
        _full((1, H)), _full((H, H)), _full((1, H)), _full((1, H)),
    ],
    out_specs=_embed_out_specs(),
    out_shape=_embed_outs(),
)


def _tc_econst_body(attr_ref, wemb_ref, bemb_ref, wes_ref, aes_ref, ec_ref):
    a = attr_ref[...]
    for l in range(3):
        we = wes_ref[pl.ds(64 * l, 64), :]
        v1 = jnp.dot(wemb_ref[...], we, preferred_element_type=F32)
        v0 = jnp.dot(bemb_ref[...], we, preferred_element_type=F32)
        ael = aes_ref[pl.ds(l, 1), :]
        ec_ref[pl.ds(l, 1), :] = jnp.full((1, 128), jnp.sum(v1 * ael), F32)
        ec_ref[pl.ds(3 + l, 1), :] = jnp.full((1, 128), jnp.sum(v0 * ael), F32)
    ec_ref[pl.ds(6, 1), :] = jnp.full((1, 128), jnp.min(a), F32)
    ec_ref[pl.ds(7, 1), :] = jnp.full((1, 128), jnp.max(a), F32)


_tc_econst = pl.pallas_call(
    _tc_econst_body,
    grid=(1,),
    in_specs=[
        _full((6250, 128)),
        _full((1, H)), _full((1, H)), _full((192, H)), _full((3, H)),
    ],
    out_specs=pl.BlockSpec((8, 128), lambda g: (0, 0)),
    out_shape=jax.ShapeDtypeStruct((8, 128), F32),
)


def _tc_wellprep_body(wx_ref, ww_ref, bw_ref, wW_ref, wad_ref,
                      sdw_ref, mxdw_ref):
    hw = jnp.dot(wx_ref[...], ww_ref[...], preferred_element_type=F32)
    hw = hw + bw_ref[...]
    hd = jnp.dot(hw, wW_ref[...], preferred_element_type=F32)
    sd_t = lax.dot_general(wad_ref[...], hd, (((1,), (1,)), ((), ())),
                           preferred_element_type=F32)
    sdw_ref[...] = sd_t
    rid = lax.broadcasted_iota(I32, (1, NWPAD), 1)
    mxdw_ref[...] = jnp.full(
        (1, 128), jnp.max(jnp.where(rid < NWELL, sd_t, -1e30)), F32)


_tc_wellprep = pl.pallas_call(
    _tc_wellprep_body,
    grid=(1,),
    in_specs=[
        _full((NWPAD, 32)), _full((32, H)), _full((1, H)),
        _full((H, H)), _full((1, H)),
    ],
    out_specs=(
        pl.BlockSpec((1, NWPAD), lambda g: (0, 0)),
        pl.BlockSpec((1, 128), lambda g: (0, 0)),
    ),
    out_shape=(
        jax.ShapeDtypeStruct((1, NWPAD), F32),
        jax.ShapeDtypeStruct((1, 128), F32),
    ),
)


def _tc_final_body(hwg_ref, m1_ref, mb1_ref, m2_ref, mb2_ref, out_ref):
    hw = jnp.concatenate(
        [hwg_ref[0], hwg_ref[1], hwg_ref[2], hwg_ref[3]], axis=1)
    z = jnp.dot(hw, m1_ref[...], preferred_element_type=F32) + mb1_ref[...]
    z = jnp.maximum(z, 0.0)
    out_ref[...] = jnp.dot(z, m2_ref[...], preferred_element_type=F32) + mb2_ref[...]


_tc_final = pl.pallas_call(
    _tc_final_body,
    grid=(1,),
    in_specs=[
        _full((4, NWPAD, 16)), _full((H, H)), _full((1, H)),
        _full((H, 75)), _full((1, 75)),
    ],
    out_specs=pl.BlockSpec((NWPAD, 75), lambda g: (0, 0)),
    out_shape=jax.ShapeDtypeStruct((NWPAD, 75), F32),
)


_MESH = plsc.VectorSubcoreMesh(core_axis_name="c", subcore_axis_name="s")

_EPT = ECC_PAD // 16
_NCH = _EPT // 128
_NHALF = NPAD // 2
_GROW = _NHALF
_ACC = _NHALF + 64
_RPT = _NHALF // 16
_NZ = _RPT // 32


def _make_sc_conv():
    @functools.partial(
        pl.kernel, mesh=_MESH,
        compiler_params=pltpu.CompilerParams(
            use_tc_tiling_on_sc=False, needs_layout_passes=False),
        out_type=jax.ShapeDtypeStruct((4, NPAD, 16), F32),
        scratch_types=[
            pltpu.VMEM((NPAD,), F32),
            pltpu.VMEM((NPAD,), F32),
            pltpu.VMEM((128,), I32),
            pltpu.VMEM((128,), I32),
            pltpu.VMEM((128,), F32),
            pltpu.VMEM((128,), F32),
            pltpu.VMEM((128, 16), F32),
            pltpu.VMEM((64,), F32),
            pltpu.VMEM((16,), F32),
            pltpu.VMEM_SHARED((_ACC, 16), F32),
            pltpu.VMEM_SHARED((_ACC,), F32),
            pltpu.SemaphoreType.DMA,
        ],
    )
    def k(src_r, dst_r, attr_r, hs_r, ss_r, sd_r, mxs_r, mxd_r, ec_r, out_r,
          ss_t, sd_t, idxs, idxd, attrc, exs, rows, fden, buf16,
          out_sp, den_sp, sem):
        c = lax.axis_index("c")
        s = lax.axis_index("s")
        pltpu.sync_copy(ss_r, ss_t)
        pltpu.sync_copy(sd_r, sd_t)
        pltpu.sync_copy(mxs_r.at[pl.ds(0, 16)], buf16)
        vmxs = buf16[...]
        pltpu.sync_copy(mxd_r.at[pl.ds(0, 16)], buf16)
        vmxd = buf16[...]
        pltpu.sync_copy(ec_r.at[pl.ds(0, 16)], buf16)
        c1v = buf16[...]
        pltpu.sync_copy(ec_r.at[pl.ds(128, 16)], buf16)
        c0v = buf16[...]
        pltpu.sync_copy(ec_r.at[pl.ds(256, 16)], buf16)
        aminv = buf16[...]
        pltpu.sync_copy(ec_r.at[pl.ds(384, 16)], buf16)
        amaxv = buf16[...]
        craw = vmxs + vmxd + jnp.maximum(c1v * aminv, c1v * amaxv) + c0v
        cshift = jnp.maximum(craw, 0.2 * craw)
        zero16 = jnp.zeros((16,), F32)

        def zr(r, carry):
            rows[r, pl.ds(0, 16)] = zero16
            return carry
        lax.fori_loop(0, 128, zr, 0)
        for g in range(8):
            exs[pl.ds(16 * g, 16)] = zero16
        zb = s * _RPT
        ebase = s * _EPT
        fb = s * _RPT

        for ph in range(4):
            q = c + 2 * (ph % 2)
            qoff = q * NPAD
            hbase = (ph // 2) * _NHALF
            do_den = (ph % 2) == 0

            def zo(i, carry):
                pltpu.sync_copy(rows.at[pl.ds(0, 32)],
                                out_sp.at[pl.ds(zb + i * 32, 32)])
                if do_den:
                    pltpu.sync_copy(exs.at[pl.ds(0, 32)],
                                    den_sp.at[pl.ds(zb + i * 32, 32)])
                return carry
            lax.fori_loop(0, _NZ, zo, 0)

            @pl.when(s == 0)
            def _():
                pltpu.sync_copy(rows.at[pl.ds(0, 64)],
                                out_sp.at[pl.ds(_NHALF, 64)])
                if do_den:
                    pltpu.sync_copy(exs.at[pl.ds(0, 64)],
                                    den_sp.at[pl.ds(_NHALF, 64)])
            plsc.subcore_barrier()

            def chunk(ci, carry):
                base = ebase + ci * 128
                pltpu.sync_copy(src_r.at[pl.ds(base, 128)], idxs)
                pltpu.sync_copy(dst_r.at[pl.ds(base, 128)], idxd)
                pltpu.sync_copy(attr_r.at[pl.ds(base, 128)], attrc)
                for g in range(8):
                    sv = idxs[pl.ds(16 * g, 16)]
                    dv = idxd[pl.ds(16 * g, 16)]
                    ssv = plsc.load_gather(ss_t, [sv])
                    sdv = plsc.load_gather(sd_t, [dv])
                    av = attrc[pl.ds(16 * g, 16)]
                    raw = ssv + sdv + av * c1v + c0v
                    alpha = jnp.maximum(raw, 0.2 * raw)
                    ex = jnp.exp(alpha - cshift)
                    eid = base + 16 * g + lax.iota(I32, 16)
                    ex = jnp.where(eid < ECC, ex, 0.0)
                    exs[pl.ds(16 * g, 16)] = ex
                    idxs[pl.ds(16 * g, 16)] = sv + qoff
                    dloc = dv - hbase
                    inh = (dloc >= 0) & (dloc < _NHALF)
                    idxd[pl.ds(16 * g, 16)] = jnp.where(inh, dloc, _GROW)
                if do_den:
                    pltpu.sync_copy(exs, den_sp.at[idxd], add=True)
                pltpu.async_copy(hs_r.at[idxs], rows, sem).wait()

                def rm(gi, carry2):
                    for j in range(16):
                        r = 16 * gi + j
                        iv = jnp.zeros((16,), I32) + r
                        ev = plsc.load_gather(exs, [iv])
                        rows[r, pl.ds(0, 16)] = rows[r, pl.ds(0, 16)] * ev
                    return carry2
                lax.fori_loop(0, 8, rm, 0)
                pltpu.sync_copy(rows, out_sp.at[idxd], add=True)
                return carry
            lax.fori_loop(0, _NCH, chunk, 0)
            plsc.subcore_barrier()

            def fl(i, carry):
                row = fb + i * 32
                pltpu.sync_copy(den_sp.at[pl.ds(row, 32)],
                                fden.at[pl.ds(0, 32)])
                pltpu.sync_copy(out_sp.at[pl.ds(row, 32)],
                                rows.at[pl.ds(0, 32)])

                def dv_(r, carry2):
                    iv = jnp.zeros((16,), I32) + r
                    dvv = plsc.load_gather(fden, [iv])
                    m = dvv > 0.0
                    v0 = rows[r, pl.ds(0, 16)]
                    rows[r, pl.ds(0, 16)] = jnp.where(m, v0 / dvv, 0.0)
                    return carry2
                lax.fori_loop(0, 32, dv_, 0)
                pltpu.sync_copy(rows.at[pl.ds(0, 32)],
                                out_r.at[q, pl.ds(hbase + row, 32)])
                return carry
            lax.fori_loop(0, _NZ, fl, 0)
            if ph < 3:
                plsc.subcore_barrier()

                def zrr(r, carry):
                    rows[r, pl.ds(0, 16)] = zero16
                    return carry
                lax.fori_loop(0, 128, zrr, 0)
                for g in range(8):
                    exs[pl.ds(16 * g, 16)] = zero16

    return k


_sc_conv = _make_sc_conv()

_EPTW = ECW_PAD // 16
_NCHW = _EPTW // 128
_WRPT = NWPAD // 16


@functools.partial(
    pl.kernel, mesh=_MESH,
    compiler_params=pltpu.CompilerParams(
        use_tc_tiling_on_sc=False, needs_layout_passes=False),
    out_type=jax.ShapeDtypeStruct((4, NWPAD, 16), F32),
    scratch_types=[
        pltpu.VMEM((NPAD,), F32),
        pltpu.VMEM((NWPAD,), F32),
        pltpu.VMEM((128,), I32),
        pltpu.VMEM((128,), I32),
        pltpu.VMEM((128,), I32),
        pltpu.VMEM((128,), F32),
        pltpu.VMEM((4, 128, 16), F32),
        pltpu.VMEM((64,), F32),
        pltpu.VMEM((16,), F32),
        pltpu.VMEM_SHARED((4, NWPAD, 16), F32),
        pltpu.VMEM_SHARED((NWPAD,), F32),
        pltpu.SemaphoreType.DMA,
    ],
)
def _sc_wells(src_r, dst_r, hs_r, ss_r, sd_r, mxs_r, mxd_r, wb_r, out_r,
              ss_t, sd_t, idxr, idxq, idxd, exs, rows, fden,
              buf16, out_sp, den_sp, sem):
    c = lax.axis_index("c")
    s = lax.axis_index("s")

    @pl.when(c == 0)
    def _():
        pltpu.sync_copy(ss_r, ss_t)
        pltpu.sync_copy(sd_r, sd_t)
        pltpu.sync_copy(mxs_r.at[pl.ds(0, 16)], buf16)
        vmxs = buf16[...]
        pltpu.sync_copy(mxd_r.at[pl.ds(0, 16)], buf16)
        vmxd = buf16[...]
        wbv = []
        for q in range(4):
            pltpu.sync_copy(wb_r.at[pl.ds(16 * q, 16)], buf16)
            wbv.append(buf16[...])
        craw = vmxs + vmxd
        cshift = jnp.maximum(craw, 0.2 * craw)
        zero16 = jnp.zeros((16,), F32)

        def zr(r, carry):
            for q in range(4):
                rows[q, r, pl.ds(0, 16)] = zero16
            return carry
        lax.fori_loop(0, 128, zr, 0)
        for g in range(8):
            exs[pl.ds(16 * g, 16)] = zero16
        zb = s * _WRPT
        for q in range(4):
            pltpu.sync_copy(rows.at[q, pl.ds(0, _WRPT)],
                            out_sp.at[q, pl.ds(zb, _WRPT)])
        pltpu.sync_copy(exs.at[pl.ds(0, _WRPT)], den_sp.at[pl.ds(zb, _WRPT)])
        plsc.subcore_barrier()

        ebase = s * _EPTW

        def chunk(ci, carry):
            base = ebase + ci * 128
            pltpu.sync_copy(src_r.at[pl.ds(base, 128)], idxr)
            pltpu.sync_copy(dst_r.at[pl.ds(base, 128)], idxd)
            for g in range(8):
                sv = idxr[pl.ds(16 * g, 16)]
                dv = idxd[pl.ds(16 * g, 16)]
                ssv = plsc.load_gather(ss_t, [sv])
                sdv = plsc.load_gather(sd_t, [dv])
                raw = ssv + sdv
                alpha = jnp.maximum(raw, 0.2 * raw)
                ex = jnp.exp(alpha - cshift)
                eid = base + 16 * g + lax.iota(I32, 16)
                ex = jnp.where(eid < ECW, ex, 0.0)
                exs[pl.ds(16 * g, 16)] = ex
            pltpu.sync_copy(exs, den_sp.at[idxd], add=True)
            for q in range(4):
                for g in range(8):
                    idxq[pl.ds(16 * g, 16)] = (
                        idxr[pl.ds(16 * g, 16)] + q * NPAD)
                pltpu.async_copy(hs_r.at[idxq], rows.at[q], sem).wait()

            def rm(r, carry2):
                iv = jnp.zeros((16,), I32) + r
                ev = plsc.load_gather(exs, [iv])
                for q in range(4):
                    rows[q, r, pl.ds(0, 16)] = rows[q, r, pl.ds(0, 16)] * ev
                return carry2
            lax.fori_loop(0, 128, rm, 0)
            for q in range(4):
                pltpu.sync_copy(rows.at[q], out_sp.at[q].at[idxd], add=True)
            return carry
        lax.fori_loop(0, _NCHW, chunk, 0)
        plsc.subcore_barrier()

        fb = s * _WRPT
        pltpu.sync_copy(den_sp.at[pl.ds(fb, _WRPT)], fden.at[pl.ds(0, _WRPT)])
        for q in range(4):
            pltpu.sync_copy(out_sp.at[q, pl.ds(fb, _WRPT)],
                            rows.at[q, pl.ds(0, _WRPT)])

        def dv_(r, carry2):
            iv = jnp.zeros((16,), I32) + r
            dvv = plsc.load_gather(fden, [iv])
            m = dvv > 0.0
            for q in range(4):
                vq = rows[q, r, pl.ds(0, 16)]
                rows[q, r, pl.ds(0, 16)] = jnp.where(m, vq / dvv, 0.0) + wbv[q]
            return carry2
        lax.fori_loop(0, _WRPT, dv_, 0)
        for q in range(4):
            pltpu.sync_copy(rows.at[q, pl.ds(0, _WRPT)],
                            out_r.at[q, pl.ds(fb, _WRPT)])


def kernel(cell_x, well_x, c2c_edge_attr, params, c2c_edge_index,
           c2w_src, c2w_dst):
    p = params
    cx = jnp.zeros((NPAD, 128), F32).at[:NCELL].set(cell_x.astype(F32))
    wx = jnp.zeros((NWPAD, 32), F32).at[:NWELL].set(well_x.astype(F32))
    attr_flat = c2c_edge_attr.astype(F32)[:, 0]
    attr2 = attr_flat.reshape(6250, 128)
    src = c2c_edge_index[0].astype(I32)
    dst = c2c_edge_index[1].astype(I32)
    srcp = jnp.pad(src, (0, ECC_PAD - ECC))
    dstp = jnp.pad(dst, (0, ECC_PAD - ECC))
    attrp = jnp.pad(attr_flat, (0, ECC_PAD - ECC))
    wsrcp = jnp.pad(c2w_src.astype(I32), (0, ECW_PAD - ECW))
    wdstp = jnp.pad(c2w_dst.astype(I32), (0, ECW_PAD - ECW))

    wemb = p["W_eemb"].astype(F32).reshape(1, H)
    bemb = p["b_eemb"].astype(F32).reshape(1, H)
    wes = jnp.concatenate([c["We"].astype(F32) for c in p["convs"]], axis=0)
    aes = jnp.stack([c["a_e"].astype(F32) for c in p["convs"]], axis=0)
    ec = _tc_econst(attr2, wemb, bemb, wes, aes)
    ec_l = [
        jnp.concatenate(
            [ec[l:l + 1], ec[3 + l:4 + l], ec[6:7], ec[7:8]], axis=0
        ).reshape(512)
        for l in range(3)
    ]

    convs = p["convs"]
    x0 = _tc_h0(cx, p["W_cell"].astype(F32),
                p["b_cell"].astype(F32).reshape(1, H))

    flags = jnp.stack([jnp.full((1, 128), v, F32) for v in (0.0, 1.0, 1.0)])
    bprevs = jnp.stack([jnp.zeros((1, H), F32),
                        convs[0]["b"].astype(F32).reshape(1, H),
                        convs[1]["b"].astype(F32).reshape(1, H)])
    ws = jnp.stack([c["W"].astype(F32) for c in convs])
    ass_ = jnp.stack([c["a_s"].astype(F32).reshape(1, H) for c in convs])
    ads_ = jnp.stack([c["a_d"].astype(F32).reshape(1, H) for c in convs])
    ecs = jnp.stack(ec_l)

    def _layer(i, x):
        flag = lax.dynamic_index_in_dim(flags, i, 0, keepdims=False)
        bprev = lax.dynamic_index_in_dim(bprevs, i, 0, keepdims=False)
        w = lax.dynamic_index_in_dim(ws, i, 0, keepdims=False)
        a_s = lax.dynamic_index_in_dim(ass_, i, 0, keepdims=False)
        a_d = lax.dynamic_index_in_dim(ads_, i, 0, keepdims=False)
        ecl = lax.dynamic_index_in_dim(ecs, i, 0, keepdims=False)
        hs, ss, sd, mxs, mxd = _tc_embed(x, flag, bprev, w, a_s, a_d)
        return _sc_conv(
            srcp, dstp, attrp, hs.reshape(4 * NPAD, 16),
            ss.reshape(NPAD), sd.reshape(NPAD),
            mxs.reshape(128), mxd.reshape(128), ecl)

    n_layers = jnp.minimum(srcp[0] * 0 + 3, 3)
    out = lax.fori_loop(0, n_layers, _layer, x0)

    hsw, ssw, _, mxsw, _ = _tc_embed(
        out, jnp.full((1, 128), 1.0, F32),
        convs[2]["b"].astype(F32).reshape(1, H),
        p["wW"].astype(F32),
        p["wa_s"].astype(F32).reshape(1, H),
        p["wa_d"].astype(F32).reshape(1, H))

    sdw, mxdw = _tc_wellprep(
        wx, p["W_well"].astype(F32), p["b_well"].astype(F32).reshape(1, H),
        p["wW"].astype(F32), p["wa_d"].astype(F32).reshape(1, H))
    hwg = _sc_wells(
        wsrcp, wdstp, hsw.reshape(4 * NPAD, 16), ssw.reshape(NPAD),
        sdw.reshape(NWPAD), mxsw.reshape(128), mxdw.reshape(128),
        p["wb"].astype(F32))

    out75 = _tc_final(
        hwg, p["m1"].astype(F32), p["mb1"].astype(F32).reshape(1, H),
        p["m2"].astype(F32), p["mb2"].astype(F32).reshape(1, 75))
    return out75[:NWELL].reshape(NWELL, 3, 25)


# speedup vs baseline: 6.2510x; 1.2636x over previous
import functools

import jax
import jax.numpy as jnp
from jax import lax
from jax.experimental import pallas as pl
from jax.experimental.pallas import tpu as pltpu
from jax.experimental.pallas import tpu_sc as plsc

F32 = jnp.float32
I32 = jnp.int32

NCELL = 50000
NPAD = 50176
NBLK = 98
BLK = 512
NWELL = 500
NWPAD = 512
ECC = 800000
ECC_PAD = 16 * 50176
ECW = 32000
ECW_PAD = 32768
H = 64


def _tc_h0_body(x_ref, wpre_ref, bpre_ref, h0_ref):
    h = jnp.dot(x_ref[...], wpre_ref[...], preferred_element_type=F32)
    h = h + bpre_ref[...]
    for q in range(4):
        h0_ref[q, :, :] = h[:, 16 * q:16 * q + 16]


def _tc_embed_body(x_ref, flag_ref, bprev_ref, w_ref, as_ref, ad_ref,
                   hs_ref, ss_ref, sd_ref, mxs_ref, mxd_ref):
    g = pl.program_id(0)
    x = jnp.concatenate([x_ref[0], x_ref[1], x_ref[2], x_ref[3]], axis=1)
    f = flag_ref[0, 0]
    h = jnp.where(f > 0.0, jnp.maximum(x + bprev_ref[...], 0.0), x)
    _tc_tail(g, h, w_ref, as_ref, ad_ref, hs_ref, ss_ref, sd_ref,
             mxs_ref, mxd_ref)


def _tc_tail(g, h, w_ref, as_ref, ad_ref, hs_ref, ss_ref, sd_ref,
             mxs_ref, mxd_ref):
    hs = jnp.dot(h, w_ref[...], preferred_element_type=F32)
    for q in range(4):
        hs_ref[q, :, :] = hs[:, 16 * q:16 * q + 16]
    ss_t = lax.dot_general(as_ref[...], hs, (((1,), (1,)), ((), ())),
                           preferred_element_type=F32)
    sd_t = lax.dot_general(ad_ref[...], hs, (((1,), (1,)), ((), ())),
                           preferred_element_type=F32)
    ss_ref[...] = ss_t.reshape(1, 1, BLK)
    sd_ref[...] = sd_t.reshape(1, 1, BLK)
    rid = g * BLK + lax.broadcasted_iota(I32, (1, BLK), 1)
    valid = rid < NCELL
    ssm = jnp.max(jnp.where(valid, ss_t, -1e30))
    sdm = jnp.max(jnp.where(valid, sd_t, -1e30))

    @pl.when(g == 0)
    def _():
        mxs_ref[...] = jnp.full((1, 128), -1e30, F32)
        mxd_ref[...] = jnp.full((1, 128), -1e30, F32)

    mxs_ref[...] = jnp.maximum(mxs_ref[...], ssm)
    mxd_ref[...] = jnp.maximum(mxd_ref[...], sdm)


def _embed_outs():
    return (
        jax.ShapeDtypeStruct((4, NPAD, 16), F32),
        jax.ShapeDtypeStruct((NBLK, 1, BLK), F32),
        jax.ShapeDtypeStruct((NBLK, 1, BLK), F32),
        jax.ShapeDtypeStruct((1, 128), F32),
        jax.ShapeDtypeStruct((1, 128), F32),
    )


def _embed_out_specs():
    return (
        pl.BlockSpec((4, BLK, 16), lambda g: (0, g, 0)),
        pl.BlockSpec((1, 1, BLK), lambda g: (g, 0, 0)),
        pl.BlockSpec((1, 1, BLK), lambda g: (g, 0, 0)),
        pl.BlockSpec((1, 128), lambda g: (0, 0)),
        pl.BlockSpec((1, 128), lambda g: (0, 0)),
    )


def _full(shape):
    return pl.BlockSpec(shape, lambda g: tuple(0 for _ in shape))


_tc_h0 = pl.pallas_call(
    _tc_h0_body,
    grid=(NBLK,),
    in_specs=[
        pl.BlockSpec((BLK, 128), lambda g: (g, 0)),
        _full((128, H)), _full((1, H)),
    ],
    out_specs=pl.BlockSpec((4, BLK, 16), lambda g: (0, g, 0)),
    out_shape=jax.ShapeDtypeStruct((4, NPAD, 16), F32),
)

_tc_embed = pl.pallas_call(
    _tc_embed_body,
    grid=(NBLK,),
    in_specs=[
        pl.BlockSpec((4, BLK, 16), lambda g: (0, g, 0)),
        _full((1, 128)),
        _full((1, H)), _full((H, H)), _full((1, H)), _full((1, H)),
    ],
    out_specs=_embed_out_specs(),
    out_shape=_embed_outs(),
)


def _tc_econst_body(attr_ref, wemb_ref, bemb_ref, wes_ref, aes_ref, ec_ref):
    a = attr_ref[...]
    for l in range(3):
        we = wes_ref[pl.ds(64 * l, 64), :]
        v1 = jnp.dot(wemb_ref[...], we, preferred_element_type=F32)
        v0 = jnp.dot(bemb_ref[...], we, preferred_element_type=F32)
        ael = aes_ref[pl.ds(l, 1), :]
        ec_ref[pl.ds(l, 1), :] = jnp.full((1, 128), jnp.sum(v1 * ael), F32)
        ec_ref[pl.ds(3 + l, 1), :] = jnp.full((1, 128), jnp.sum(v0 * ael), F32)
    ec_ref[pl.ds(6, 1), :] = jnp.full((1, 128), jnp.min(a), F32)
    ec_ref[pl.ds(7, 1), :] = jnp.full((1, 128), jnp.max(a), F32)


_tc_econst = pl.pallas_call(
    _tc_econst_body,
    grid=(1,),
    in_specs=[
        _full((6250, 128)),
        _full((1, H)), _full((1, H)), _full((192, H)), _full((3, H)),
    ],
    out_specs=pl.BlockSpec((8, 128), lambda g: (0, 0)),
    out_shape=jax.ShapeDtypeStruct((8, 128), F32),
)


def _tc_wellprep_body(wx_ref, ww_ref, bw_ref, wW_ref, wad_ref,
                      sdw_ref, mxdw_ref):
    hw = jnp.dot(wx_ref[...], ww_ref[...], preferred_element_type=F32)
    hw = hw + bw_ref[...]
    hd = jnp.dot(hw, wW_ref[...], preferred_element_type=F32)
    sd_t = lax.dot_general(wad_ref[...], hd, (((1,), (1,)), ((), ())),
                           preferred_element_type=F32)
    sdw_ref[...] = sd_t
    rid = lax.broadcasted_iota(I32, (1, NWPAD), 1)
    mxdw_ref[...] = jnp.full(
        (1, 128), jnp.max(jnp.where(rid < NWELL, sd_t, -1e30)), F32)


_tc_wellprep = pl.pallas_call(
    _tc_wellprep_body,
    grid=(1,),
    in_specs=[
        _full((NWPAD, 32)), _full((32, H)), _full((1, H)),
        _full((H, H)), _full((1, H)),
    ],
    out_specs=(
        pl.BlockSpec((1, NWPAD), lambda g: (0, 0)),
        pl.BlockSpec((1, 128), lambda g: (0, 0)),
    ),
    out_shape=(
        jax.ShapeDtypeStruct((1, NWPAD), F32),
        jax.ShapeDtypeStruct((1, 128), F32),
    ),
)


def _tc_final_body(outp_ref, denp_ref, wb_ref, m1_ref, mb1_ref, m2_ref,
                   mb2_ref, out_ref):
    hw = jnp.sum(outp_ref[...], axis=0)
    den_row = jnp.sum(denp_ref[...], axis=0).reshape(1, NWPAD)
    ii = lax.broadcasted_iota(I32, (NWPAD, NWPAD), 0)
    jj = lax.broadcasted_iota(I32, (NWPAD, NWPAD), 1)
    ident = jnp.where(ii == jj, 1.0, 0.0).astype(F32)
    den_col = lax.dot_general(ident, den_row, (((1,), (1,)), ((), ())),
                              preferred_element_type=F32)
    hwn = jnp.where(den_col > 0.0, hw / den_col, 0.0) + wb_ref[...]
    z = jnp.dot(hwn, m1_ref[...], preferred_element_type=F32) + mb1_ref[...]
    z = jnp.maximum(z, 0.0)
    out_ref[...] = jnp.dot(z, m2_ref[...], preferred_element_type=F32) + mb2_ref[...]


_tc_final = pl.pallas_call(
    _tc_final_body,
    grid=(1,),
    in_specs=[
        _full((32, NWPAD, 64)), _full((32, NWPAD)), _full((1, H)),
        _full((H, H)), _full((1, H)),
        _full((H, 75)), _full((1, 75)),
    ],
    out_specs=pl.BlockSpec((NWPAD, 75), lambda g: (0, 0)),
    out_shape=jax.ShapeDtypeStruct((NWPAD, 75), F32),
)


_MESH = plsc.VectorSubcoreMesh(core_axis_name="c", subcore_axis_name="s")

_EPT = ECC_PAD // 16
_CH = 128
_NCH = _EPT // _CH
_NHALF = NPAD // 2
_GROW = _NHALF
_ACC = _NHALF + 8
_RPT = _NHALF // 16
_NZ = _RPT // 32


def _make_sc_conv():
    @functools.partial(
        pl.kernel, mesh=_MESH,
        compiler_params=pltpu.CompilerParams(
            use_tc_tiling_on_sc=False, needs_layout_passes=False),
        out_type=jax.ShapeDtypeStruct((4, NPAD, 16), F32),
        scratch_types=[
            pltpu.VMEM((NPAD,), F32),
            pltpu.VMEM((NPAD,), F32),
            pltpu.VMEM((3 * _CH,), I32),
            pltpu.VMEM((_CH,), I32),
            pltpu.VMEM((_CH,), I32),
            pltpu.VMEM((_CH,), F32),
            pltpu.VMEM((_CH, 16), F32),
            pltpu.VMEM((64,), F32),
            pltpu.VMEM((16,), F32),
            pltpu.VMEM_SHARED((_ACC, 16), F32),
            pltpu.VMEM_SHARED((_ACC,), F32),
            pltpu.SemaphoreType.DMA,
        ],
    )
    def k(epk_r, hs_r, ss_r, sd_r, mxs_r, mxd_r, ec_r, out_r,
          ss_t, sd_t, ebuf, idxs, idxd, exs, rows, fden, buf16,
          out_sp, den_sp, sem):
        c = lax.axis_index("c")
        s = lax.axis_index("s")
        pltpu.sync_copy(ss_r, ss_t)
        pltpu.sync_copy(sd_r, sd_t)
        pltpu.sync_copy(mxs_r.at[pl.ds(0, 16)], buf16)
        vmxs = buf16[...]
        pltpu.sync_copy(mxd_r.at[pl.ds(0, 16)], buf16)
        vmxd = buf16[...]
        pltpu.sync_copy(ec_r.at[pl.ds(0, 16)], buf16)
        c1v = buf16[...]
        pltpu.sync_copy(ec_r.at[pl.ds(128, 16)], buf16)
        c0v = buf16[...]
        pltpu.sync_copy(ec_r.at[pl.ds(256, 16)], buf16)
        aminv = buf16[...]
        pltpu.sync_copy(ec_r.at[pl.ds(384, 16)], buf16)
        amaxv = buf16[...]
        craw = vmxs + vmxd + jnp.maximum(c1v * aminv, c1v * amaxv) + c0v
        cshift = jnp.maximum(craw, 0.2 * craw)
        zero16 = jnp.zeros((16,), F32)

        def zr(r, carry):
            rows[r, pl.ds(0, 16)] = zero16
            return carry
        lax.fori_loop(0, 128, zr, 0)
        for g in range(8):
            exs[pl.ds(16 * g, 16)] = zero16
        zb = s * _RPT
        ebase = s * _EPT
        fb = s * _RPT

        for ph in range(4):
            q = c + 2 * (ph % 2)
            qoff = q * NPAD
            hbase = (ph // 2) * _NHALF
            do_den = (ph % 2) == 0

            def zo(i, carry):
                pltpu.sync_copy(rows.at[pl.ds(0, 32)],
                                out_sp.at[pl.ds(zb + i * 32, 32)])
                if do_den:
                    pltpu.sync_copy(exs.at[pl.ds(0, 32)],
                                    den_sp.at[pl.ds(zb + i * 32, 32)])
                return carry
            lax.fori_loop(0, _NZ, zo, 0)

            @pl.when(s == 0)
            def _():
                pltpu.sync_copy(rows.at[pl.ds(0, 8)],
                                out_sp.at[pl.ds(_NHALF, 8)])
                if do_den:
                    pltpu.sync_copy(exs.at[pl.ds(0, 8)],
                                    den_sp.at[pl.ds(_NHALF, 8)])
            plsc.subcore_barrier()

            def chunk(ci, carry):
                base = ebase + ci * _CH
                pltpu.sync_copy(epk_r.at[pl.ds(base * 3, 3 * _CH)], ebuf)
                for g in range(_CH // 16):
                    sv = ebuf[pl.ds(16 * g, 16)]
                    dv = ebuf[pl.ds(_CH + 16 * g, 16)]
                    ssv = plsc.load_gather(ss_t, [sv])
                    sdv = plsc.load_gather(sd_t, [dv])
                    av = plsc.bitcast(ebuf[pl.ds(2 * _CH + 16 * g, 16)], F32)
                    raw = ssv + sdv + av * c1v + c0v
                    alpha = jnp.maximum(raw, 0.2 * raw)
                    ex = jnp.exp(alpha - cshift)
                    eid = base + 16 * g + lax.iota(I32, 16)
                    ex = jnp.where(eid < ECC, ex, 0.0)
                    exs[pl.ds(16 * g, 16)] = ex
                    idxs[pl.ds(16 * g, 16)] = sv + qoff
                    dloc = dv - hbase
                    inh = (dloc >= 0) & (dloc < _NHALF)
                    idxd[pl.ds(16 * g, 16)] = jnp.where(inh, dloc, _GROW)
                if do_den:
                    pltpu.sync_copy(exs, den_sp.at[idxd], add=True)
                pltpu.async_copy(hs_r.at[idxs], rows, sem).wait()

                def rm(r, carry2):
                    iv = jnp.zeros((16,), I32) + r
                    ev = plsc.load_gather(exs, [iv])
                    rows[r, pl.ds(0, 16)] = rows[r, pl.ds(0, 16)] * ev
                    return carry2
                lax.fori_loop(0, _CH, rm, 0)
                pltpu.sync_copy(rows, out_sp.at[idxd], add=True)
                return carry
            lax.fori_loop(0, _NCH, chunk, 0)
            plsc.subcore_barrier()

            def fl(i, carry):
                row = fb + i * 32
                pltpu.sync_copy(den_sp.at[pl.ds(row, 32)],
                                fden.at[pl.ds(0, 32)])
                pltpu.sync_copy(out_sp.at[pl.ds(row, 32)],
                                rows.at[pl.ds(0, 32)])

                def dv_(r, carry2):
                    iv = jnp.zeros((16,), I32) + r
                    dvv = plsc.load_gather(fden, [iv])
                    m = dvv > 0.0
                    v0 = rows[r, pl.ds(0, 16)]
                    rows[r, pl.ds(0, 16)] = jnp.where(m, v0 / dvv, 0.0)
                    return carry2
                lax.fori_loop(0, 32, dv_, 0)
                pltpu.sync_copy(rows.at[pl.ds(0, 32)],
                                out_r.at[q, pl.ds(hbase + row, 32)])
                return carry
            lax.fori_loop(0, _NZ, fl, 0)
            if ph < 3:
                plsc.subcore_barrier()

                def zrr(r, carry):
                    rows[r, pl.ds(0, 16)] = zero16
                    return carry
                lax.fori_loop(0, 128, zrr, 0)
                for g in range(8):
                    exs[pl.ds(16 * g, 16)] = zero16

    return k


_sc_conv = _make_sc_conv()

_EPTW = ECW_PAD // 16
_NCHW = _EPTW // 128
_WRPT = NWPAD // 16


@functools.partial(
    pl.kernel, mesh=_MESH,
    compiler_params=pltpu.CompilerParams(
        use_tc_tiling_on_sc=False, needs_layout_passes=False),
    out_type=(
        jax.ShapeDtypeStruct((32, NWPAD * 64), F32),
        jax.ShapeDtypeStruct((32, NWPAD), F32),
    ),
    scratch_types=[
        pltpu.VMEM((NPAD,), F32),
        pltpu.VMEM((NWPAD,), F32),
        pltpu.VMEM((128,), I32),
        pltpu.VMEM((128,), I32),
        pltpu.VMEM((128,), I32),
        pltpu.VMEM((128,), F32),
        pltpu.VMEM((4, 128, 16), F32),
        pltpu.VMEM((NWPAD * 64,), F32),
        pltpu.VMEM((NWPAD,), F32),
        pltpu.VMEM((16,), F32),
        pltpu.SemaphoreType.DMA,
    ],
)
def _sc_wells(src_r, dst_r, hs_r, ss_r, sd_r, mxs_r, mxd_r, out_r, den_r,
              ss_t, sd_t, idxr, idxq, idxd, exs, rows, acc_t, den_t,
              buf16, sem):
    c = lax.axis_index("c")
    s = lax.axis_index("s")
    w = s * 2 + c
    pltpu.sync_copy(ss_r, ss_t)
    pltpu.sync_copy(sd_r, sd_t)
    pltpu.sync_copy(mxs_r.at[pl.ds(0, 16)], buf16)
    vmxs = buf16[...]
    pltpu.sync_copy(mxd_r.at[pl.ds(0, 16)], buf16)
    vmxd = buf16[...]
    craw = vmxs + vmxd
    cshift = jnp.maximum(craw, 0.2 * craw)
    zero16 = jnp.zeros((16,), F32)

    def zacc(r, carry):
        for q in range(4):
            acc_t[pl.ds(r * 64 + 16 * q, 16)] = zero16
        return carry
    lax.fori_loop(0, NWPAD, zacc, 0)
    for g in range(NWPAD // 16):
        den_t[pl.ds(16 * g, 16)] = zero16
    lane = lax.iota(I32, 16)
    mask0 = lane == 0

    ebase = w * (ECW_PAD // 32)

    def chunk(ci, carry):
        base = ebase + ci * 128
        pltpu.sync_copy(src_r.at[pl.ds(base, 128)], idxr)
        pltpu.sync_copy(dst_r.at[pl.ds(base, 128)], idxd)
        for g in range(8):
            sv = idxr[pl.ds(16 * g, 16)]
            dv = idxd[pl.ds(16 * g, 16)]
            ssv = plsc.load_gather(ss_t, [sv])
            sdv = plsc.load_gather(sd_t, [dv])
            raw = ssv + sdv
            alpha = jnp.maximum(raw, 0.2 * raw)
            ex = jnp.exp(alpha - cshift)
            eid = base + 16 * g + lax.iota(I32, 16)
            ex = jnp.where(eid < ECW, ex, 0.0)
            exs[pl.ds(16 * g, 16)] = ex
        for q in range(4):
            for g in range(8):
                idxq[pl.ds(16 * g, 16)] = (
                    idxr[pl.ds(16 * g, 16)] + q * NPAD)
            pltpu.async_copy(hs_r.at[idxq], rows.at[q], sem).wait()

        def rm(r, carry2):
            iv = jnp.zeros((16,), I32) + r
            ev = plsc.load_gather(exs, [iv])
            d16 = plsc.load_gather(idxd, [iv])
            plsc.addupdate_scatter(den_t, [d16], ev, mask=mask0)
            fbase = d16 * 64 + lane
            for q in range(4):
                rv = rows[q, r, pl.ds(0, 16)] * ev
                plsc.addupdate_scatter(acc_t, [fbase + 16 * q], rv)
            return carry2
        lax.fori_loop(0, 128, rm, 0)
        return carry
    lax.fori_loop(0, ECW_PAD // 32 // 128, chunk, 0)
    pltpu.sync_copy(acc_t, out_r.at[w])
    pltpu.sync_copy(den_t, den_r.at[w])


def kernel(cell_x, well_x, c2c_edge_attr, params, c2c_edge_index,
           c2w_src, c2w_dst):
    p = params
    cx = jnp.zeros((NPAD, 128), F32).at[:NCELL].set(cell_x.astype(F32))
    wx = jnp.zeros((NWPAD, 32), F32).at[:NWELL].set(well_x.astype(F32))
    attr_flat = c2c_edge_attr.astype(F32)[:, 0]
    attr2 = attr_flat.reshape(6250, 128)
    src = c2c_edge_index[0].astype(I32)
    dst = c2c_edge_index[1].astype(I32)
    srcp = jnp.pad(src, (0, ECC_PAD - ECC))
    dstp = jnp.pad(dst, (0, ECC_PAD - ECC))
    attrp = jnp.pad(attr_flat, (0, ECC_PAD - ECC))
    attrb = lax.bitcast_convert_type(attrp, I32)
    epk = jnp.stack(
        [srcp.reshape(-1, 128), dstp.reshape(-1, 128),
         attrb.reshape(-1, 128)], axis=1).reshape(3 * ECC_PAD)
    wsrcp = jnp.pad(c2w_src.astype(I32), (0, ECW_PAD - ECW))
    wdstp = jnp.pad(c2w_dst.astype(I32), (0, ECW_PAD - ECW))

    wemb = p["W_eemb"].astype(F32).reshape(1, H)
    bemb = p["b_eemb"].astype(F32).reshape(1, H)
    wes = jnp.concatenate([c["We"].astype(F32) for c in p["convs"]], axis=0)
    aes = jnp.stack([c["a_e"].astype(F32) for c in p["convs"]], axis=0)
    ec = _tc_econst(attr2, wemb, bemb, wes, aes)
    ec_l = [
        jnp.concatenate(
            [ec[l:l + 1], ec[3 + l:4 + l], ec[6:7], ec[7:8]], axis=0
        ).reshape(512)
        for l in range(3)
    ]

    convs = p["convs"]
    x0 = _tc_h0(cx, p["W_cell"].astype(F32),
                p["b_cell"].astype(F32).reshape(1, H))

    flags = jnp.stack([jnp.full((1, 128), v, F32) for v in (0.0, 1.0, 1.0)])
    bprevs = jnp.stack([jnp.zeros((1, H), F32),
                        convs[0]["b"].astype(F32).reshape(1, H),
                        convs[1]["b"].astype(F32).reshape(1, H)])
    ws = jnp.stack([c["W"].astype(F32) for c in convs])
    ass_ = jnp.stack([c["a_s"].astype(F32).reshape(1, H) for c in convs])
    ads_ = jnp.stack([c["a_d"].astype(F32).reshape(1, H) for c in convs])
    ecs = jnp.stack(ec_l)

    def _layer(i, x):
        flag = lax.dynamic_index_in_dim(flags, i, 0, keepdims=False)
        bprev = lax.dynamic_index_in_dim(bprevs, i, 0, keepdims=False)
        w = lax.dynamic_index_in_dim(ws, i, 0, keepdims=False)
        a_s = lax.dynamic_index_in_dim(ass_, i, 0, keepdims=False)
        a_d = lax.dynamic_index_in_dim(ads_, i, 0, keepdims=False)
        ecl = lax.dynamic_index_in_dim(ecs, i, 0, keepdims=False)
        hs, ss, sd, mxs, mxd = _tc_embed(x, flag, bprev, w, a_s, a_d)
        return _sc_conv(
            epk, hs.reshape(4 * NPAD, 16),
            ss.reshape(NPAD), sd.reshape(NPAD),
            mxs.reshape(128), mxd.reshape(128), ecl)

    n_layers = jnp.minimum(srcp[0] * 0 + 3, 3)
    out = lax.fori_loop(0, n_layers, _layer, x0)

    hsw, ssw, _, mxsw, _ = _tc_embed(
        out, jnp.full((1, 128), 1.0, F32),
        convs[2]["b"].astype(F32).reshape(1, H),
        p["wW"].astype(F32),
        p["wa_s"].astype(F32).reshape(1, H),
        p["wa_d"].astype(F32).reshape(1, H))

    sdw, mxdw = _tc_wellprep(
        wx, p["W_well"].astype(F32), p["b_well"].astype(F32).reshape(1, H),
        p["wW"].astype(F32), p["wa_d"].astype(F32).reshape(1, H))
    outp, denp = _sc_wells(
        wsrcp, wdstp, hsw.reshape(4 * NPAD, 16), ssw.reshape(NPAD),
        sdw.reshape(NWPAD), mxsw.reshape(128), mxdw.reshape(128))

    out75 = _tc_final(
        outp.reshape(32, NWPAD, 64), denp,
        p["wb"].astype(F32).reshape(1, H),
        p["m1"].astype(F32), p["mb1"].astype(F32).reshape(1, H),
        p["m2"].astype(F32), p["mb2"].astype(F32).reshape(1, 75))
    return out75[:NWELL].reshape(NWELL, 3, 25)


# speedup vs baseline: 6.3318x; 1.0129x over previous
import functools

import jax
import jax.numpy as jnp
from jax import lax
from jax.experimental import pallas as pl
from jax.experimental.pallas import tpu as pltpu
from jax.experimental.pallas import tpu_sc as plsc

F32 = jnp.float32
I32 = jnp.int32

NCELL = 50000
NPAD = 50176
NBLK = 98
BLK = 512
NWELL = 500
NWPAD = 512
ECC = 800000
ECC_PAD = 16 * 50176
ECW = 32000
ECW_PAD = 32768
H = 64


def _tc_h0_body(x_ref, wpre_ref, bpre_ref, h0_ref):
    h = jnp.dot(x_ref[...], wpre_ref[...], preferred_element_type=F32)
    h = h + bpre_ref[...]
    for q in range(4):
        h0_ref[q, :, :] = h[:, 16 * q:16 * q + 16]


def _tc_embed_body(x_ref, den_ref, flag_ref, bprev_ref, w_ref, as_ref,
                   ad_ref, hs_ref, ss_ref, sd_ref, mxs_ref, mxd_ref):
    g = pl.program_id(0)
    x = jnp.concatenate([x_ref[0], x_ref[1], x_ref[2], x_ref[3]], axis=1)
    f = flag_ref[0, 0]
    den_row = den_ref[...].reshape(1, BLK)
    ii = lax.broadcasted_iota(I32, (BLK, BLK), 0)
    jj = lax.broadcasted_iota(I32, (BLK, BLK), 1)
    ident = jnp.where(ii == jj, 1.0, 0.0).astype(F32)
    den_col = lax.dot_general(ident, den_row, (((1,), (1,)), ((), ())),
                              preferred_element_type=F32)
    xn = jnp.where(den_col > 0.0, x / den_col, 0.0)
    h = jnp.where(f > 0.0, jnp.maximum(xn + bprev_ref[...], 0.0), x)
    _tc_tail(g, h, w_ref, as_ref, ad_ref, hs_ref, ss_ref, sd_ref,
             mxs_ref, mxd_ref)


def _tc_tail(g, h, w_ref, as_ref, ad_ref, hs_ref, ss_ref, sd_ref,
             mxs_ref, mxd_ref):
    hs = jnp.dot(h, w_ref[...], preferred_element_type=F32)
    for q in range(4):
        hs_ref[q, :, :] = hs[:, 16 * q:16 * q + 16]
    ss_t = lax.dot_general(as_ref[...], hs, (((1,), (1,)), ((), ())),
                           preferred_element_type=F32)
    sd_t = lax.dot_general(ad_ref[...], hs, (((1,), (1,)), ((), ())),
                           preferred_element_type=F32)
    ss_ref[...] = ss_t.reshape(1, 1, BLK)
    sd_ref[...] = sd_t.reshape(1, 1, BLK)
    rid = g * BLK + lax.broadcasted_iota(I32, (1, BLK), 1)
    valid = rid < NCELL
    ssm = jnp.max(jnp.where(valid, ss_t, -1e30))
    sdm = jnp.max(jnp.where(valid, sd_t, -1e30))

    @pl.when(g == 0)
    def _():
        mxs_ref[...] = jnp.full((1, 128), -1e30, F32)
        mxd_ref[...] = jnp.full((1, 128), -1e30, F32)

    mxs_ref[...] = jnp.maximum(mxs_ref[...], ssm)
    mxd_ref[...] = jnp.maximum(mxd_ref[...], sdm)


def _embed_outs():
    return (
        jax.ShapeDtypeStruct((4, NPAD, 16), F32),
        jax.ShapeDtypeStruct((NBLK, 1, BLK), F32),
        jax.ShapeDtypeStruct((NBLK, 1, BLK), F32),
        jax.ShapeDtypeStruct((1, 128), F32),
        jax.ShapeDtypeStruct((1, 128), F32),
    )


def _embed_out_specs():
    return (
        pl.BlockSpec((4, BLK, 16), lambda g: (0, g, 0)),
        pl.BlockSpec((1, 1, BLK), lambda g: (g, 0, 0)),
        pl.BlockSpec((1, 1, BLK), lambda g: (g, 0, 0)),
        pl.BlockSpec((1, 128), lambda g: (0, 0)),
        pl.BlockSpec((1, 128), lambda g: (0, 0)),
    )


def _full(shape):
    return pl.BlockSpec(shape, lambda g: tuple(0 for _ in shape))


_tc_h0 = pl.pallas_call(
    _tc_h0_body,
    grid=(NBLK,),
    in_specs=[
        pl.BlockSpec((BLK, 128), lambda g: (g, 0)),
        _full((128, H)), _full((1, H)),
    ],
    out_specs=pl.BlockSpec((4, BLK, 16), lambda g: (0, g, 0)),
    out_shape=jax.ShapeDtypeStruct((4, NPAD, 16), F32),
)

_tc_embed = pl.pallas_call(
    _tc_embed_body,
    grid=(NBLK,),
    in_specs=[
        pl.BlockSpec((4, BLK, 16), lambda g: (0, g, 0)),
        pl.BlockSpec((1, 1, BLK), lambda g: (g, 0, 0)),
        _full((1, 128)),
        _full((1, H)), _full((H, H)), _full((1, H)), _full((1, H)),
    ],
    out_specs=_embed_out_specs(),
    out_shape=_embed_outs(),
)


def _tc_econst_body(attr_ref, wemb_ref, bemb_ref, wes_ref, aes_ref, ec_ref):
    a = attr_ref[...]
    for l in range(3):
        we = wes_ref[pl.ds(64 * l, 64), :]
        v1 = jnp.dot(wemb_ref[...], we, preferred_element_type=F32)
        v0 = jnp.dot(bemb_ref[...], we, preferred_element_type=F32)
        ael = aes_ref[pl.ds(l, 1), :]
        ec_ref[pl.ds(l, 1), :] = jnp.full((1, 128), jnp.sum(v1 * ael), F32)
        ec_ref[pl.ds(3 + l, 1), :] = jnp.full((1, 128), jnp.sum(v0 * ael), F32)
    ec_ref[pl.ds(6, 1), :] = jnp.full((1, 128), jnp.min(a), F32)
    ec_ref[pl.ds(7, 1), :] = jnp.full((1, 128), jnp.max(a), F32)


_tc_econst = pl.pallas_call(
    _tc_econst_body,
    grid=(1,),
    in_specs=[
        _full((6250, 128)),
        _full((1, H)), _full((1, H)), _full((192, H)), _full((3, H)),
    ],
    out_specs=pl.BlockSpec((8, 128), lambda g: (0, 0)),
    out_shape=jax.ShapeDtypeStruct((8, 128), F32),
)


def _tc_wellprep_body(wx_ref, ww_ref, bw_ref, wW_ref, wad_ref,
                      sdw_ref, mxdw_ref):
    hw = jnp.dot(wx_ref[...], ww_ref[...], preferred_element_type=F32)
    hw = hw + bw_ref[...]
    hd = jnp.dot(hw, wW_ref[...], preferred_element_type=F32)
    sd_t = lax.dot_general(wad_ref[...], hd, (((1,), (1,)), ((), ())),
                           preferred_element_type=F32)
    sdw_ref[...] = sd_t
    rid = lax.broadcasted_iota(I32, (1, NWPAD), 1)
    mxdw_ref[...] = jnp.full(
        (1, 128), jnp.max(jnp.where(rid < NWELL, sd_t, -1e30)), F32)


_tc_wellprep = pl.pallas_call(
    _tc_wellprep_body,
    grid=(1,),
    in_specs=[
        _full((NWPAD, 32)), _full((32, H)), _full((1, H)),
        _full((H, H)), _full((1, H)),
    ],
    out_specs=(
        pl.BlockSpec((1, NWPAD), lambda g: (0, 0)),
        pl.BlockSpec((1, 128), lambda g: (0, 0)),
    ),
    out_shape=(
        jax.ShapeDtypeStruct((1, NWPAD), F32),
        jax.ShapeDtypeStruct((1, 128), F32),
    ),
)


def _tc_final_body(outp_ref, denp_ref, wb_ref, m1_ref, mb1_ref, m2_ref,
                   mb2_ref, out_ref):
    hw = jnp.sum(outp_ref[...], axis=0)
    den_row = jnp.sum(denp_ref[...], axis=0).reshape(1, NWPAD)
    ii = lax.broadcasted_iota(I32, (NWPAD, NWPAD), 0)
    jj = lax.broadcasted_iota(I32, (NWPAD, NWPAD), 1)
    ident = jnp.where(ii == jj, 1.0, 0.0).astype(F32)
    den_col = lax.dot_general(ident, den_row, (((1,), (1,)), ((), ())),
                              preferred_element_type=F32)
    hwn = jnp.where(den_col > 0.0, hw / den_col, 0.0) + wb_ref[...]
    z = jnp.dot(hwn, m1_ref[...], preferred_element_type=F32) + mb1_ref[...]
    z = jnp.maximum(z, 0.0)
    out_ref[...] = jnp.dot(z, m2_ref[...], preferred_element_type=F32) + mb2_ref[...]


_tc_final = pl.pallas_call(
    _tc_final_body,
    grid=(1,),
    in_specs=[
        _full((32, NWPAD, 64)), _full((32, NWPAD)), _full((1, H)),
        _full((H, H)), _full((1, H)),
        _full((H, 75)), _full((1, 75)),
    ],
    out_specs=pl.BlockSpec((NWPAD, 75), lambda g: (0, 0)),
    out_shape=jax.ShapeDtypeStruct((NWPAD, 75), F32),
)


_MESH = plsc.VectorSubcoreMesh(core_axis_name="c", subcore_axis_name="s")

_EPT = ECC_PAD // 16
_CH = 128
_NCH = _EPT // _CH
_NHALF = NPAD // 2
_GROW = _NHALF
_ACC = _NHALF
_RPT = _NHALF // 16
_NZ = _RPT // 32


def _make_sc_conv():
    @functools.partial(
        pl.kernel, mesh=_MESH,
        compiler_params=pltpu.CompilerParams(
            use_tc_tiling_on_sc=False, needs_layout_passes=False),
        out_type=(
            jax.ShapeDtypeStruct((4, NPAD, 16), F32),
            jax.ShapeDtypeStruct((2, _NHALF), F32),
        ),
        scratch_types=[
            pltpu.VMEM((NPAD,), F32),
            pltpu.VMEM((NPAD,), F32),
            pltpu.VMEM((3 * _CH,), I32),
            pltpu.VMEM((_CH,), I32),
            pltpu.VMEM((_CH,), I32),
            pltpu.VMEM((_CH,), F32),
            pltpu.VMEM((_CH, 16), F32),
            pltpu.VMEM((16,), F32),
            pltpu.VMEM_SHARED((_ACC, 16), F32),
            pltpu.VMEM_SHARED((_ACC,), F32),
            pltpu.SemaphoreType.DMA,
        ],
    )
    def k(epk_r, hs_r, ss_r, sd_r, mxs_r, mxd_r, ec_r, out_r, den_r,
          ss_t, sd_t, ebuf0, idxs0, idxd0, exs0, rows0, buf16,
          out_sp, den_sp, sem0):
        c = lax.axis_index("c")
        s = lax.axis_index("s")

        def tl(i, carry):
            pltpu.sync_copy(ss_r.at[pl.ds(i * 6272, 6272)],
                            ss_t.at[pl.ds(i * 6272, 6272)])
            pltpu.sync_copy(sd_r.at[pl.ds(i * 6272, 6272)],
                            sd_t.at[pl.ds(i * 6272, 6272)])
            return carry
        lax.fori_loop(0, NPAD // 6272, tl, 0)
        pltpu.sync_copy(mxs_r.at[pl.ds(0, 16)], buf16)
        vmxs = buf16[...]
        pltpu.sync_copy(mxd_r.at[pl.ds(0, 16)], buf16)
        vmxd = buf16[...]
        pltpu.sync_copy(ec_r.at[pl.ds(0, 16)], buf16)
        c1v = buf16[...]
        pltpu.sync_copy(ec_r.at[pl.ds(128, 16)], buf16)
        c0v = buf16[...]
        pltpu.sync_copy(ec_r.at[pl.ds(256, 16)], buf16)
        aminv = buf16[...]
        pltpu.sync_copy(ec_r.at[pl.ds(384, 16)], buf16)
        amaxv = buf16[...]
        craw = vmxs + vmxd + jnp.maximum(c1v * aminv, c1v * amaxv) + c0v
        cshift = jnp.maximum(craw, 0.2 * craw)
        zero16 = jnp.zeros((16,), F32)

        def zr(r, carry):
            rows0[r, pl.ds(0, 16)] = zero16
            return carry
        lax.fori_loop(0, _CH, zr, 0)
        for g in range(_CH // 16):
            exs0[pl.ds(16 * g, 16)] = zero16
        zb = s * _RPT
        ebase = s * _EPT
        fb = s * _RPT

        def phase(ph, carry0):
            q = c + 2 * (ph % 2)
            qoff = q * NPAD
            hbase = (ph // 2) * _NHALF
            do_den = (ph % 2) == 0

            def zo(i, carry):
                pltpu.sync_copy(rows0.at[pl.ds(0, 32)],
                                out_sp.at[pl.ds(zb + i * 32, 32)])

                @pl.when(do_den)
                def _():
                    pltpu.sync_copy(exs0.at[pl.ds(0, 32)],
                                    den_sp.at[pl.ds(zb + i * 32, 32)])
                return carry
            lax.fori_loop(0, _NZ, zo, 0)

            plsc.subcore_barrier()

            def start_chunk(ci, eb, ixs, ixd, exb, rb, semx):
                base = ebase + ci * _CH
                pltpu.sync_copy(epk_r.at[pl.ds(base * 3, 3 * _CH)], eb)
                for g in range(_CH // 16):
                    sv = eb[pl.ds(16 * g, 16)]
                    dv = eb[pl.ds(_CH + 16 * g, 16)]
                    ssv = plsc.load_gather(ss_t, [sv])
                    sdv = plsc.load_gather(sd_t, [dv])
                    av = plsc.bitcast(eb[pl.ds(2 * _CH + 16 * g, 16)], F32)
                    raw = ssv + sdv + av * c1v + c0v
                    alpha = jnp.maximum(raw, 0.2 * raw)
                    ex = jnp.exp(alpha - cshift)
                    eid = base + 16 * g + lax.iota(I32, 16)
                    dloc = dv - hbase
                    inh = (dloc >= 0) & (dloc < _NHALF) & (eid < ECC)
                    exb[pl.ds(16 * g, 16)] = jnp.where(inh, ex, 0.0)
                    ixs[pl.ds(16 * g, 16)] = sv + qoff
                    ixd[pl.ds(16 * g, 16)] = jnp.where(inh, dloc, 0)

                @pl.when(do_den)
                def _():
                    pltpu.sync_copy(exb, den_sp.at[ixd], add=True)
                pltpu.make_async_copy(hs_r.at[ixs], rb, semx).start()

            def finish_chunk(ixs, ixd, exb, rb, semx):
                pltpu.make_async_copy(hs_r.at[ixs], rb, semx).wait()

                def rm(r, carry2):
                    iv = jnp.zeros((16,), I32) + r
                    ev = plsc.load_gather(exb, [iv])
                    rb[r, pl.ds(0, 16)] = rb[r, pl.ds(0, 16)] * ev
                    return carry2
                lax.fori_loop(0, _CH, rm, 0)
                pltpu.sync_copy(rb, out_sp.at[ixd], add=True)

            def pipe(j, carry):
                start_chunk(j, ebuf0, idxs0, idxd0, exs0, rows0, sem0)
                finish_chunk(idxs0, idxd0, exs0, rows0, sem0)
                return carry
            lax.fori_loop(0, _NCH, pipe, 0)
            plsc.subcore_barrier()

            pltpu.sync_copy(out_sp.at[pl.ds(fb, _RPT)],
                            out_r.at[q, pl.ds(hbase + fb, _RPT)])

            @pl.when(do_den & (c == 0))
            def _():
                pltpu.sync_copy(den_sp.at[pl.ds(fb, _RPT)],
                                den_r.at[ph // 2, pl.ds(fb, _RPT)])
            plsc.subcore_barrier()

            def zrr(r, carry):
                rows0[r, pl.ds(0, 16)] = zero16
                return carry
            lax.fori_loop(0, _CH, zrr, 0)
            for g in range(_CH // 16):
                exs0[pl.ds(16 * g, 16)] = zero16
            return carry0

        lax.fori_loop(0, 4, phase, 0)

    return k


_sc_conv = _make_sc_conv()

_EPTW = ECW_PAD // 16
_NCHW = _EPTW // 128
_WRPT = NWPAD // 16


@functools.partial(
    pl.kernel, mesh=_MESH,
    compiler_params=pltpu.CompilerParams(
        use_tc_tiling_on_sc=False, needs_layout_passes=False),
    out_type=(
        jax.ShapeDtypeStruct((32, NWPAD * 64), F32),
        jax.ShapeDtypeStruct((32, NWPAD), F32),
    ),
    scratch_types=[
        pltpu.VMEM((NPAD,), F32),
        pltpu.VMEM((NWPAD,), F32),
        pltpu.VMEM((128,), I32),
        pltpu.VMEM((128,), I32),
        pltpu.VMEM((128,), I32),
        pltpu.VMEM((128,), F32),
        pltpu.VMEM((4, 128, 16), F32),
        pltpu.VMEM((NWPAD * 64,), F32),
        pltpu.VMEM((NWPAD,), F32),
        pltpu.VMEM((16,), F32),
        pltpu.SemaphoreType.DMA,
    ],
)
def _sc_wells(src_r, dst_r, hs_r, ss_r, sd_r, mxs_r, mxd_r, out_r, den_r,
              ss_t, sd_t, idxr, idxq, idxd, exs, rows, acc_t, den_t,
              buf16, sem):
    c = lax.axis_index("c")
    s = lax.axis_index("s")
    w = s * 2 + c
    pltpu.sync_copy(ss_r, ss_t)
    pltpu.sync_copy(sd_r, sd_t)
    pltpu.sync_copy(mxs_r.at[pl.ds(0, 16)], buf16)
    vmxs = buf16[...]
    pltpu.sync_copy(mxd_r.at[pl.ds(0, 16)], buf16)
    vmxd = buf16[...]
    craw = vmxs + vmxd
    cshift = jnp.maximum(craw, 0.2 * craw)
    zero16 = jnp.zeros((16,), F32)

    def zacc(r, carry):
        for q in range(4):
            acc_t[pl.ds(r * 64 + 16 * q, 16)] = zero16
        return carry
    lax.fori_loop(0, NWPAD, zacc, 0)
    for g in range(NWPAD // 16):
        den_t[pl.ds(16 * g, 16)] = zero16
    lane = lax.iota(I32, 16)
    mask0 = lane == 0

    ebase = w * (ECW_PAD // 32)

    def chunk(ci, carry):
        base = ebase + ci * 128
        pltpu.sync_copy(src_r.at[pl.ds(base, 128)], idxr)
        pltpu.sync_copy(dst_r.at[pl.ds(base, 128)], idxd)
        for g in range(8):
            sv = idxr[pl.ds(16 * g, 16)]
            dv = idxd[pl.ds(16 * g, 16)]
            ssv = plsc.load_gather(ss_t, [sv])
            sdv = plsc.load_gather(sd_t, [dv])
            raw = ssv + sdv
            alpha = jnp.maximum(raw, 0.2 * raw)
            ex = jnp.exp(alpha - cshift)
            eid = base + 16 * g + lax.iota(I32, 16)
            ex = jnp.where(eid < ECW, ex, 0.0)
            exs[pl.ds(16 * g, 16)] = ex
        for q in range(4):
            for g in range(8):
                idxq[pl.ds(16 * g, 16)] = (
                    idxr[pl.ds(16 * g, 16)] + q * NPAD)
            pltpu.async_copy(hs_r.at[idxq], rows.at[q], sem).wait()

        def rm(r, carry2):
            iv = jnp.zeros((16,), I32) + r
            ev = plsc.load_gather(exs, [iv])
            d16 = plsc.load_gather(idxd, [iv])
            plsc.addupdate_scatter(den_t, [d16], ev, mask=mask0)
            fbase = d16 * 64 + lane
            for q in range(4):
                rv = rows[q, r, pl.ds(0, 16)] * ev
                plsc.addupdate_scatter(acc_t, [fbase + 16 * q], rv)
            return carry2
        lax.fori_loop(0, 128, rm, 0)
        return carry
    lax.fori_loop(0, ECW_PAD // 32 // 128, chunk, 0)
    pltpu.sync_copy(acc_t, out_r.at[w])
    pltpu.sync_copy(den_t, den_r.at[w])


def kernel(cell_x, well_x, c2c_edge_attr, params, c2c_edge_index,
           c2w_src, c2w_dst):
    p = params
    cx = jnp.zeros((NPAD, 128), F32).at[:NCELL].set(cell_x.astype(F32))
    wx = jnp.zeros((NWPAD, 32), F32).at[:NWELL].set(well_x.astype(F32))
    attr_flat = c2c_edge_attr.astype(F32)[:, 0]
    attr2 = attr_flat.reshape(6250, 128)
    src = c2c_edge_index[0].astype(I32)
    dst = c2c_edge_index[1].astype(I32)
    srcp = jnp.pad(src, (0, ECC_PAD - ECC))
    dstp = jnp.pad(dst, (0, ECC_PAD - ECC))
    attrp = jnp.pad(attr_flat, (0, ECC_PAD - ECC))
    attrb = lax.bitcast_convert_type(attrp, I32)
    epk = jnp.stack(
        [srcp.reshape(-1, 128), dstp.reshape(-1, 128),
         attrb.reshape(-1, 128)], axis=1).reshape(3 * ECC_PAD)
    wsrcp = jnp.pad(c2w_src.astype(I32), (0, ECW_PAD - ECW))
    wdstp = jnp.pad(c2w_dst.astype(I32), (0, ECW_PAD - ECW))

    wemb = p["W_eemb"].astype(F32).reshape(1, H)
    bemb = p["b_eemb"].astype(F32).reshape(1, H)
    wes = jnp.concatenate([c["We"].astype(F32) for c in p["convs"]], axis=0)
    aes = jnp.stack([c["a_e"].astype(F32) for c in p["convs"]], axis=0)
    ec = _tc_econst(attr2, wemb, bemb, wes, aes)
    ec_l = [
        jnp.concatenate(
            [ec[l:l + 1], ec[3 + l:4 + l], ec[6:7], ec[7:8]], axis=0
        ).reshape(512)
        for l in range(3)
    ]

    convs = p["convs"]
    x0 = _tc_h0(cx, p["W_cell"].astype(F32),
                p["b_cell"].astype(F32).reshape(1, H))

    flags = jnp.stack([jnp.full((1, 128), v, F32) for v in (0.0, 1.0, 1.0)])
    bprevs = jnp.stack([jnp.zeros((1, H), F32),
                        convs[0]["b"].astype(F32).reshape(1, H),
                        convs[1]["b"].astype(F32).reshape(1, H)])
    ws = jnp.stack([c["W"].astype(F32) for c in convs])
    ass_ = jnp.stack([c["a_s"].astype(F32).reshape(1, H) for c in convs])
    ads_ = jnp.stack([c["a_d"].astype(F32).reshape(1, H) for c in convs])
    ecs = jnp.stack(ec_l)

    def _layer(i, carry):
        x, den = carry
        flag = lax.dynamic_index_in_dim(flags, i, 0, keepdims=False)
        bprev = lax.dynamic_index_in_dim(bprevs, i, 0, keepdims=False)
        w = lax.dynamic_index_in_dim(ws, i, 0, keepdims=False)
        a_s = lax.dynamic_index_in_dim(ass_, i, 0, keepdims=False)
        a_d = lax.dynamic_index_in_dim(ads_, i, 0, keepdims=False)
        ecl = lax.dynamic_index_in_dim(ecs, i, 0, keepdims=False)
        hs, ss, sd, mxs, mxd = _tc_embed(
            x, den.reshape(NBLK, 1, BLK), flag, bprev, w, a_s, a_d)
        o, dn = _sc_conv(
            epk, hs.reshape(4 * NPAD, 16),
            ss.reshape(NPAD), sd.reshape(NPAD),
            mxs.reshape(128), mxd.reshape(128), ecl)
        return (o, dn.reshape(NPAD))

    n_layers = jnp.minimum(srcp[0] * 0 + 3, 3)
    out, den3 = lax.fori_loop(0, n_layers, _layer,
                              (x0, jnp.ones((NPAD,), F32)))

    hsw, ssw, _, mxsw, _ = _tc_embed(
        out, den3.reshape(NBLK, 1, BLK), jnp.full((1, 128), 1.0, F32),
        convs[2]["b"].astype(F32).reshape(1, H),
        p["wW"].astype(F32),
        p["wa_s"].astype(F32).reshape(1, H),
        p["wa_d"].astype(F32).reshape(1, H))

    sdw, mxdw = _tc_wellprep(
        wx, p["W_well"].astype(F32), p["b_well"].astype(F32).reshape(1, H),
        p["wW"].astype(F32), p["wa_d"].astype(F32).reshape(1, H))
    outp, denp = _sc_wells(
        wsrcp, wdstp, hsw.reshape(4 * NPAD, 16), ssw.reshape(NPAD),
        sdw.reshape(NWPAD), mxsw.reshape(128), mxdw.reshape(128))

    out75 = _tc_final(
        outp.reshape(32, NWPAD, 64), denp,
        p["wb"].astype(F32).reshape(1, H),
        p["m1"].astype(F32), p["mb1"].astype(F32).reshape(1, H),
        p["m2"].astype(F32), p["mb2"].astype(F32).reshape(1, 75))
    return out75[:NWELL].reshape(NWELL, 3, 25)


# speedup vs baseline: 6.4570x; 1.0198x over previous
import functools

import jax
import jax.numpy as jnp
from jax import lax
from jax.experimental import pallas as pl
from jax.experimental.pallas import tpu as pltpu
from jax.experimental.pallas import tpu_sc as plsc

F32 = jnp.float32
I32 = jnp.int32

NCELL = 50000
NPAD = 50176
NBLK = 98
BLK = 512
NWELL = 500
NWPAD = 512
ECC = 800000
ECC_PAD = 16 * 50176
ECW = 32000
ECW_PAD = 32768
H = 64


def _tc_h0_body(x_ref, wpre_ref, bpre_ref, h0_ref):
    h = jnp.dot(x_ref[...], wpre_ref[...], preferred_element_type=F32)
    h = h + bpre_ref[...]
    for q in range(4):
        h0_ref[q, :, :] = h[:, 16 * q:16 * q + 16]


def _tc_embed_body(x_ref, den_ref, flag_ref, bprev_ref, w_ref, as_ref,
                   ad_ref, hs_ref, ss_ref, sd_ref, mxs_ref, mxd_ref):
    g = pl.program_id(0)
    x = jnp.concatenate([x_ref[0], x_ref[1], x_ref[2], x_ref[3]], axis=1)
    f = flag_ref[0, 0]
    den_row = den_ref[...].reshape(1, BLK)
    ii = lax.broadcasted_iota(I32, (BLK, BLK), 0)
    jj = lax.broadcasted_iota(I32, (BLK, BLK), 1)
    ident = jnp.where(ii == jj, 1.0, 0.0).astype(F32)
    den_col = lax.dot_general(ident, den_row, (((1,), (1,)), ((), ())),
                              preferred_element_type=F32)
    xn = jnp.where(den_col > 0.0, x / den_col, 0.0)
    h = jnp.where(f > 0.0, jnp.maximum(xn + bprev_ref[...], 0.0), x)
    _tc_tail(g, h, w_ref, as_ref, ad_ref, hs_ref, ss_ref, sd_ref,
             mxs_ref, mxd_ref)


def _tc_tail(g, h, w_ref, as_ref, ad_ref, hs_ref, ss_ref, sd_ref,
             mxs_ref, mxd_ref):
    hs = jnp.dot(h, w_ref[...], preferred_element_type=F32)
    for q in range(4):
        hs_ref[q, :, :] = hs[:, 16 * q:16 * q + 16]
    ss_t = lax.dot_general(as_ref[...], hs, (((1,), (1,)), ((), ())),
                           preferred_element_type=F32)
    sd_t = lax.dot_general(ad_ref[...], hs, (((1,), (1,)), ((), ())),
                           preferred_element_type=F32)
    ss_ref[...] = ss_t.reshape(1, 1, BLK)
    sd_ref[...] = sd_t.reshape(1, 1, BLK)
    rid = g * BLK + lax.broadcasted_iota(I32, (1, BLK), 1)
    valid = rid < NCELL
    ssm = jnp.max(jnp.where(valid, ss_t, -1e30))
    sdm = jnp.max(jnp.where(valid, sd_t, -1e30))

    @pl.when(g == 0)
    def _():
        mxs_ref[...] = jnp.full((1, 128), -1e30, F32)
        mxd_ref[...] = jnp.full((1, 128), -1e30, F32)

    mxs_ref[...] = jnp.maximum(mxs_ref[...], ssm)
    mxd_ref[...] = jnp.maximum(mxd_ref[...], sdm)


def _embed_outs():
    return (
        jax.ShapeDtypeStruct((4, NPAD, 16), F32),
        jax.ShapeDtypeStruct((NBLK, 1, BLK), F32),
        jax.ShapeDtypeStruct((NBLK, 1, BLK), F32),
        jax.ShapeDtypeStruct((1, 128), F32),
        jax.ShapeDtypeStruct((1, 128), F32),
    )


def _embed_out_specs():
    return (
        pl.BlockSpec((4, BLK, 16), lambda g: (0, g, 0)),
        pl.BlockSpec((1, 1, BLK), lambda g: (g, 0, 0)),
        pl.BlockSpec((1, 1, BLK), lambda g: (g, 0, 0)),
        pl.BlockSpec((1, 128), lambda g: (0, 0)),
        pl.BlockSpec((1, 128), lambda g: (0, 0)),
    )


def _full(shape):
    return pl.BlockSpec(shape, lambda g: tuple(0 for _ in shape))


_tc_h0 = pl.pallas_call(
    _tc_h0_body,
    grid=(NBLK,),
    in_specs=[
        pl.BlockSpec((BLK, 128), lambda g: (g, 0)),
        _full((128, H)), _full((1, H)),
    ],
    out_specs=pl.BlockSpec((4, BLK, 16), lambda g: (0, g, 0)),
    out_shape=jax.ShapeDtypeStruct((4, NPAD, 16), F32),
)

_tc_embed = pl.pallas_call(
    _tc_embed_body,
    grid=(NBLK,),
    in_specs=[
        pl.BlockSpec((4, BLK, 16), lambda g: (0, g, 0)),
        pl.BlockSpec((1, 1, BLK), lambda g: (g, 0, 0)),
        _full((1, 128)),
        _full((1, H)), _full((H, H)), _full((1, H)), _full((1, H)),
    ],
    out_specs=_embed_out_specs(),
    out_shape=_embed_outs(),
)


def _tc_econst_body(attr_ref, wemb_ref, bemb_ref, wes_ref, aes_ref, ec_ref):
    a = attr_ref[...]
    for l in range(3):
        we = wes_ref[pl.ds(64 * l, 64), :]
        v1 = jnp.dot(wemb_ref[...], we, preferred_element_type=F32)
        v0 = jnp.dot(bemb_ref[...], we, preferred_element_type=F32)
        ael = aes_ref[pl.ds(l, 1), :]
        ec_ref[pl.ds(l, 1), :] = jnp.full((1, 128), jnp.sum(v1 * ael), F32)
        ec_ref[pl.ds(3 + l, 1), :] = jnp.full((1, 128), jnp.sum(v0 * ael), F32)
    ec_ref[pl.ds(6, 1), :] = jnp.full((1, 128), jnp.min(a), F32)
    ec_ref[pl.ds(7, 1), :] = jnp.full((1, 128), jnp.max(a), F32)


_tc_econst = pl.pallas_call(
    _tc_econst_body,
    grid=(1,),
    in_specs=[
        _full((6250, 128)),
        _full((1, H)), _full((1, H)), _full((192, H)), _full((3, H)),
    ],
    out_specs=pl.BlockSpec((8, 128), lambda g: (0, 0)),
    out_shape=jax.ShapeDtypeStruct((8, 128), F32),
)


def _tc_wellprep_body(wx_ref, ww_ref, bw_ref, wW_ref, wad_ref,
                      sdw_ref, mxdw_ref):
    hw = jnp.dot(wx_ref[...], ww_ref[...], preferred_element_type=F32)
    hw = hw + bw_ref[...]
    hd = jnp.dot(hw, wW_ref[...], preferred_element_type=F32)
    sd_t = lax.dot_general(wad_ref[...], hd, (((1,), (1,)), ((), ())),
                           preferred_element_type=F32)
    sdw_ref[...] = sd_t
    rid = lax.broadcasted_iota(I32, (1, NWPAD), 1)
    mxdw_ref[...] = jnp.full(
        (1, 128), jnp.max(jnp.where(rid < NWELL, sd_t, -1e30)), F32)


_tc_wellprep = pl.pallas_call(
    _tc_wellprep_body,
    grid=(1,),
    in_specs=[
        _full((NWPAD, 32)), _full((32, H)), _full((1, H)),
        _full((H, H)), _full((1, H)),
    ],
    out_specs=(
        pl.BlockSpec((1, NWPAD), lambda g: (0, 0)),
        pl.BlockSpec((1, 128), lambda g: (0, 0)),
    ),
    out_shape=(
        jax.ShapeDtypeStruct((1, NWPAD), F32),
        jax.ShapeDtypeStruct((1, 128), F32),
    ),
)


def _tc_final_body(outp_ref, denp_ref, wb_ref, m1_ref, mb1_ref, m2_ref,
                   mb2_ref, out_ref):
    hw = jnp.sum(outp_ref[...], axis=0)
    den_row = jnp.sum(denp_ref[...], axis=0).reshape(1, NWPAD)
    ii = lax.broadcasted_iota(I32, (NWPAD, NWPAD), 0)
    jj = lax.broadcasted_iota(I32, (NWPAD, NWPAD), 1)
    ident = jnp.where(ii == jj, 1.0, 0.0).astype(F32)
    den_col = lax.dot_general(ident, den_row, (((1,), (1,)), ((), ())),
                              preferred_element_type=F32)
    hwn = jnp.where(den_col > 0.0, hw / den_col, 0.0) + wb_ref[...]
    z = jnp.dot(hwn, m1_ref[...], preferred_element_type=F32) + mb1_ref[...]
    z = jnp.maximum(z, 0.0)
    out_ref[...] = jnp.dot(z, m2_ref[...], preferred_element_type=F32) + mb2_ref[...]


_tc_final = pl.pallas_call(
    _tc_final_body,
    grid=(1,),
    in_specs=[
        _full((32, NWPAD, 64)), _full((32, NWPAD)), _full((1, H)),
        _full((H, H)), _full((1, H)),
        _full((H, 75)), _full((1, 75)),
    ],
    out_specs=pl.BlockSpec((NWPAD, 75), lambda g: (0, 0)),
    out_shape=jax.ShapeDtypeStruct((NWPAD, 75), F32),
)


_MESH = plsc.VectorSubcoreMesh(core_axis_name="c", subcore_axis_name="s")

_EPT = ECC_PAD // 16
_CH = 128
_NCH = _EPT // _CH
_NHALF = NPAD // 2
_GROW = _NHALF
_ACC = _NHALF
_RPT = _NHALF // 16
_NZ = _RPT // 32


def _make_sc_conv():
    @functools.partial(
        pl.kernel, mesh=_MESH,
        compiler_params=pltpu.CompilerParams(
            use_tc_tiling_on_sc=False, needs_layout_passes=False),
        out_type=(
            jax.ShapeDtypeStruct((4, NPAD, 16), F32),
            jax.ShapeDtypeStruct((2, _NHALF), F32),
        ),
        scratch_types=[
            pltpu.VMEM((2 * NPAD,), F32),
            pltpu.VMEM((3 * _CH,), I32),
            pltpu.VMEM((_CH,), I32),
            pltpu.VMEM((_CH,), I32),
            pltpu.VMEM((_CH,), I32),
            pltpu.VMEM((_CH,), I32),
            pltpu.VMEM((_CH,), F32),
            pltpu.VMEM((_CH,), F32),
            pltpu.VMEM((_CH, 16), F32),
            pltpu.VMEM((_CH, 16), F32),
            pltpu.VMEM((768,), F32),
            pltpu.VMEM_SHARED((_ACC, 16), F32),
            pltpu.VMEM_SHARED((_ACC,), F32),
            pltpu.SemaphoreType.DMA,
            pltpu.SemaphoreType.DMA,
        ],
    )
    def k(epk_r, hs_r, sstab_r, cst_r, out_r, den_r,
          ss_t, ebuf0, idxs0, idxs1, idxd0, idxd1, exs0, exs1,
          rows0, rows1, cbuf, out_sp, den_sp, sem0, sem1):
        c = lax.axis_index("c")
        s = lax.axis_index("s")

        def tl(i, carry):
            pltpu.sync_copy(sstab_r.at[pl.ds(i * 12544, 12544)],
                            ss_t.at[pl.ds(i * 12544, 12544)])
            return carry
        lax.fori_loop(0, 2 * NPAD // 12544, tl, 0)
        pltpu.sync_copy(cst_r, cbuf)
        c1v = cbuf[pl.ds(0, 16)]
        c0v = cbuf[pl.ds(128, 16)]
        aminv = cbuf[pl.ds(256, 16)]
        amaxv = cbuf[pl.ds(384, 16)]
        vmxs = cbuf[pl.ds(512, 16)]
        vmxd = cbuf[pl.ds(640, 16)]
        craw = vmxs + vmxd + jnp.maximum(c1v * aminv, c1v * amaxv) + c0v
        cshift = jnp.maximum(craw, 0.2 * craw)
        zero16 = jnp.zeros((16,), F32)

        def zr(r, carry):
            rows0[r, pl.ds(0, 16)] = zero16
            return carry
        lax.fori_loop(0, _CH, zr, 0)
        for g in range(_CH // 16):
            exs0[pl.ds(16 * g, 16)] = zero16
        zb = s * _RPT
        ebase = s * _EPT
        fb = s * _RPT

        def phase(ph, carry0):
            q = c + 2 * (ph % 2)
            qoff = q * NPAD
            hbase = (ph // 2) * _NHALF
            do_den = (ph % 2) == 0

            def zo(i, carry):
                pltpu.sync_copy(rows0.at[pl.ds(0, 32)],
                                out_sp.at[pl.ds(zb + i * 32, 32)])

                @pl.when(do_den)
                def _():
                    pltpu.sync_copy(exs0.at[pl.ds(0, 32)],
                                    den_sp.at[pl.ds(zb + i * 32, 32)])
                return carry
            lax.fori_loop(0, _NZ, zo, 0)

            plsc.subcore_barrier()

            def start_chunk(ci, eb, ixs, ixd, exb, rb, semx):
                base = ebase + ci * _CH
                pltpu.sync_copy(epk_r.at[pl.ds(base * 3, 3 * _CH)], eb)
                for g in range(_CH // 16):
                    sv = eb[pl.ds(16 * g, 16)]
                    dv = eb[pl.ds(_CH + 16 * g, 16)]
                    ssv = plsc.load_gather(ss_t, [sv])
                    sdv = plsc.load_gather(ss_t, [dv + NPAD])
                    av = plsc.bitcast(eb[pl.ds(2 * _CH + 16 * g, 16)], F32)
                    raw = ssv + sdv + av * c1v + c0v
                    alpha = jnp.maximum(raw, 0.2 * raw)
                    ex = jnp.exp(alpha - cshift)
                    eid = base + 16 * g + lax.iota(I32, 16)
                    dloc = dv - hbase
                    inh = (dloc >= 0) & (dloc < _NHALF) & (eid < ECC)
                    exb[pl.ds(16 * g, 16)] = jnp.where(inh, ex, 0.0)
                    ixs[pl.ds(16 * g, 16)] = sv + qoff
                    ixd[pl.ds(16 * g, 16)] = jnp.where(inh, dloc, 0)

                pltpu.make_async_copy(hs_r.at[ixs], rb, semx).start()

                @pl.when(do_den)
                def _():
                    pltpu.sync_copy(exb, den_sp.at[ixd], add=True)

            def finish_chunk(ixs, ixd, exb, rb, semx):
                pltpu.make_async_copy(hs_r.at[ixs], rb, semx).wait()

                def rm(r, carry2):
                    iv = jnp.zeros((16,), I32) + r
                    ev = plsc.load_gather(exb, [iv])
                    rb[r, pl.ds(0, 16)] = rb[r, pl.ds(0, 16)] * ev
                    return carry2
                lax.fori_loop(0, _CH, rm, 0)
                pltpu.sync_copy(rb, out_sp.at[ixd], add=True)

            def pipe(j, carry):
                start_chunk(j, ebuf0, idxs0, idxd0, exs0, rows0, sem0)
                finish_chunk(idxs0, idxd0, exs0, rows0, sem0)
                return carry
            lax.fori_loop(0, _NCH, pipe, 0)
            plsc.subcore_barrier()

            pltpu.sync_copy(out_sp.at[pl.ds(fb, _RPT)],
                            out_r.at[q, pl.ds(hbase + fb, _RPT)])

            @pl.when(do_den & (c == 0))
            def _():
                pltpu.sync_copy(den_sp.at[pl.ds(fb, _RPT)],
                                den_r.at[ph // 2, pl.ds(fb, _RPT)])
            plsc.subcore_barrier()

            def zrr(r, carry):
                rows0[r, pl.ds(0, 16)] = zero16
                return carry
            lax.fori_loop(0, _CH, zrr, 0)
            for g in range(_CH // 16):
                exs0[pl.ds(16 * g, 16)] = zero16
            return carry0

        lax.fori_loop(0, 4, phase, 0)

    return k


_sc_conv = _make_sc_conv()

_EPTW = ECW_PAD // 16
_NCHW = _EPTW // 128
_WRPT = NWPAD // 16


@functools.partial(
    pl.kernel, mesh=_MESH,
    compiler_params=pltpu.CompilerParams(
        use_tc_tiling_on_sc=False, needs_layout_passes=False),
    out_type=(
        jax.ShapeDtypeStruct((32, NWPAD * 64), F32),
        jax.ShapeDtypeStruct((32, NWPAD), F32),
    ),
    scratch_types=[
        pltpu.VMEM((NPAD,), F32),
        pltpu.VMEM((NWPAD,), F32),
        pltpu.VMEM((128,), I32),
        pltpu.VMEM((128,), I32),
        pltpu.VMEM((128,), I32),
        pltpu.VMEM((128,), F32),
        pltpu.VMEM((4, 128, 16), F32),
        pltpu.VMEM((NWPAD * 64,), F32),
        pltpu.VMEM((NWPAD,), F32),
        pltpu.VMEM((16,), F32),
        pltpu.SemaphoreType.DMA,
    ],
)
def _sc_wells(src_r, dst_r, hs_r, ss_r, sd_r, mxs_r, mxd_r, out_r, den_r,
              ss_t, sd_t, idxr, idxq, idxd, exs, rows, acc_t, den_t,
              buf16, sem):
    c = lax.axis_index("c")
    s = lax.axis_index("s")
    w = s * 2 + c
    pltpu.sync_copy(ss_r, ss_t)
    pltpu.sync_copy(sd_r, sd_t)
    pltpu.sync_copy(mxs_r.at[pl.ds(0, 16)], buf16)
    vmxs = buf16[...]
    pltpu.sync_copy(mxd_r.at[pl.ds(0, 16)], buf16)
    vmxd = buf16[...]
    craw = vmxs + vmxd
    cshift = jnp.maximum(craw, 0.2 * craw)
    zero16 = jnp.zeros((16,), F32)

    def zacc(r, carry):
        for q in range(4):
            acc_t[pl.ds(r * 64 + 16 * q, 16)] = zero16
        return carry
    lax.fori_loop(0, NWPAD, zacc, 0)
    for g in range(NWPAD // 16):
        den_t[pl.ds(16 * g, 16)] = zero16
    lane = lax.iota(I32, 16)
    mask0 = lane == 0

    ebase = w * (ECW_PAD // 32)

    def chunk(ci, carry):
        base = ebase + ci * 128
        pltpu.sync_copy(src_r.at[pl.ds(base, 128)], idxr)
        pltpu.sync_copy(dst_r.at[pl.ds(base, 128)], idxd)
        for g in range(8):
            sv = idxr[pl.ds(16 * g, 16)]
            dv = idxd[pl.ds(16 * g, 16)]
            ssv = plsc.load_gather(ss_t, [sv])
            sdv = plsc.load_gather(sd_t, [dv])
            raw = ssv + sdv
            alpha = jnp.maximum(raw, 0.2 * raw)
            ex = jnp.exp(alpha - cshift)
            eid = base + 16 * g + lax.iota(I32, 16)
            ex = jnp.where(eid < ECW, ex, 0.0)
            exs[pl.ds(16 * g, 16)] = ex
        for q in range(4):
            for g in range(8):
                idxq[pl.ds(16 * g, 16)] = (
                    idxr[pl.ds(16 * g, 16)] + q * NPAD)
            pltpu.async_copy(hs_r.at[idxq], rows.at[q], sem).wait()

        def rm(r, carry2):
            iv = jnp.zeros((16,), I32) + r
            ev = plsc.load_gather(exs, [iv])
            d16 = plsc.load_gather(idxd, [iv])
            plsc.addupdate_scatter(den_t, [d16], ev, mask=mask0)
            fbase = d16 * 64 + lane
            for q in range(4):
                rv = rows[q, r, pl.ds(0, 16)] * ev
                plsc.addupdate_scatter(acc_t, [fbase + 16 * q], rv)
            return carry2
        lax.fori_loop(0, 128, rm, 0)
        return carry
    lax.fori_loop(0, ECW_PAD // 32 // 128, chunk, 0)
    pltpu.sync_copy(acc_t, out_r.at[w])
    pltpu.sync_copy(den_t, den_r.at[w])


def kernel(cell_x, well_x, c2c_edge_attr, params, c2c_edge_index,
           c2w_src, c2w_dst):
    p = params
    cx = jnp.zeros((NPAD, 128), F32).at[:NCELL].set(cell_x.astype(F32))
    wx = jnp.zeros((NWPAD, 32), F32).at[:NWELL].set(well_x.astype(F32))
    attr_flat = c2c_edge_attr.astype(F32)[:, 0]
    attr2 = attr_flat.reshape(6250, 128)
    src = c2c_edge_index[0].astype(I32)
    dst = c2c_edge_index[1].astype(I32)
    srcp = jnp.pad(src, (0, ECC_PAD - ECC))
    dstp = jnp.pad(dst, (0, ECC_PAD - ECC))
    attrp = jnp.pad(attr_flat, (0, ECC_PAD - ECC))
    attrb = lax.bitcast_convert_type(attrp, I32)
    epk = jnp.stack(
        [srcp.reshape(-1, 128), dstp.reshape(-1, 128),
         attrb.reshape(-1, 128)], axis=1).reshape(3 * ECC_PAD)
    wsrcp = jnp.pad(c2w_src.astype(I32), (0, ECW_PAD - ECW))
    wdstp = jnp.pad(c2w_dst.astype(I32), (0, ECW_PAD - ECW))

    wemb = p["W_eemb"].astype(F32).reshape(1, H)
    bemb = p["b_eemb"].astype(F32).reshape(1, H)
    wes = jnp.concatenate([c["We"].astype(F32) for c in p["convs"]], axis=0)
    aes = jnp.stack([c["a_e"].astype(F32) for c in p["convs"]], axis=0)
    ec = _tc_econst(attr2, wemb, bemb, wes, aes)
    ec_l = [
        jnp.concatenate(
            [ec[l:l + 1], ec[3 + l:4 + l], ec[6:7], ec[7:8]], axis=0
        ).reshape(512)
        for l in range(3)
    ]

    convs = p["convs"]
    x0 = _tc_h0(cx, p["W_cell"].astype(F32),
                p["b_cell"].astype(F32).reshape(1, H))

    flags = jnp.stack([jnp.full((1, 128), v, F32) for v in (0.0, 1.0, 1.0)])
    bprevs = jnp.stack([jnp.zeros((1, H), F32),
                        convs[0]["b"].astype(F32).reshape(1, H),
                        convs[1]["b"].astype(F32).reshape(1, H)])
    ws = jnp.stack([c["W"].astype(F32) for c in convs])
    ass_ = jnp.stack([c["a_s"].astype(F32).reshape(1, H) for c in convs])
    ads_ = jnp.stack([c["a_d"].astype(F32).reshape(1, H) for c in convs])
    ecs = jnp.stack(ec_l)

    def _layer(i, carry):
        x, den = carry
        flag = lax.dynamic_index_in_dim(flags, i, 0, keepdims=False)
        bprev = lax.dynamic_index_in_dim(bprevs, i, 0, keepdims=False)
        w = lax.dynamic_index_in_dim(ws, i, 0, keepdims=False)
        a_s = lax.dynamic_index_in_dim(ass_, i, 0, keepdims=False)
        a_d = lax.dynamic_index_in_dim(ads_, i, 0, keepdims=False)
        ecl = lax.dynamic_index_in_dim(ecs, i, 0, keepdims=False)
        hs, ss, sd, mxs, mxd = _tc_embed(
            x, den.reshape(NBLK, 1, BLK), flag, bprev, w, a_s, a_d)
        sstab = jnp.concatenate([ss.reshape(NPAD), sd.reshape(NPAD)])
        cst = jnp.concatenate([ecl, mxs.reshape(128), mxd.reshape(128)])
        o, dn = _sc_conv(epk, hs.reshape(4 * NPAD, 16), sstab, cst)
        return (o, dn.reshape(NPAD))

    n_layers = jnp.minimum(srcp[0] * 0 + 3, 3)
    out, den3 = lax.fori_loop(0, n_layers, _layer,
                              (x0, jnp.ones((NPAD,), F32)))

    hsw, ssw, _, mxsw, _ = _tc_embed(
        out, den3.reshape(NBLK, 1, BLK), jnp.full((1, 128), 1.0, F32),
        convs[2]["b"].astype(F32).reshape(1, H),
        p["wW"].astype(F32),
        p["wa_s"].astype(F32).reshape(1, H),
        p["wa_d"].astype(F32).reshape(1, H))

    sdw, mxdw = _tc_wellprep(
        wx, p["W_well"].astype(F32), p["b_well"].astype(F32).reshape(1, H),
        p["wW"].astype(F32), p["wa_d"].astype(F32).reshape(1, H))
    outp, denp = _sc_wells(
        wsrcp, wdstp, hsw.reshape(4 * NPAD, 16), ssw.reshape(NPAD),
        sdw.reshape(NWPAD), mxsw.reshape(128), mxdw.reshape(128))

    out75 = _tc_final(
        outp.reshape(32, NWPAD, 64), denp,
        p["wb"].astype(F32).reshape(1, H),
        p["m1"].astype(F32), p["mb1"].astype(F32).reshape(1, H),
        p["m2"].astype(F32), p["mb2"].astype(F32).reshape(1, 75))
    return out75[:NWELL].reshape(NWELL, 3, 25)


# speedup vs baseline: 6.8177x; 1.0559x over previous
import functools

import jax
import jax.numpy as jnp
from jax import lax
from jax.experimental import pallas as pl
from jax.experimental.pallas import tpu as pltpu
from jax.experimental.pallas import tpu_sc as plsc

F32 = jnp.float32
I32 = jnp.int32

NCELL = 50000
NPAD = 50176
NBLK = 98
BLK = 512
NWELL = 500
NWPAD = 512
ECC = 800000
ECC_PAD = 16 * 50176
ECW = 32000
ECW_PAD = 32768
H = 64


def _tc_h0_body(x_ref, wpre_ref, bpre_ref, h0_ref):
    h = jnp.dot(x_ref[...], wpre_ref[...], preferred_element_type=F32)
    h = h + bpre_ref[...]
    for q in range(4):
        h0_ref[q, :, :] = h[:, 16 * q:16 * q + 16]


def _tc_embed_body(x_ref, den_ref, flag_ref, bprev_ref, w_ref, as_ref,
                   ad_ref, hs_ref, ss_ref, sd_ref, mxs_ref, mxd_ref):
    g = pl.program_id(0)
    x = jnp.concatenate([x_ref[0], x_ref[1], x_ref[2], x_ref[3]], axis=1)
    f = flag_ref[0, 0]
    den_row = den_ref[...].reshape(1, BLK)
    ii = lax.broadcasted_iota(I32, (BLK, BLK), 0)
    jj = lax.broadcasted_iota(I32, (BLK, BLK), 1)
    ident = jnp.where(ii == jj, 1.0, 0.0).astype(F32)
    den_col = lax.dot_general(ident, den_row, (((1,), (1,)), ((), ())),
                              preferred_element_type=F32)
    xn = jnp.where(den_col > 0.0, x / den_col, 0.0)
    h = jnp.where(f > 0.0, jnp.maximum(xn + bprev_ref[...], 0.0), x)
    _tc_tail(g, h, w_ref, as_ref, ad_ref, hs_ref, ss_ref, sd_ref,
             mxs_ref, mxd_ref)


def _tc_tail(g, h, w_ref, as_ref, ad_ref, hs_ref, ss_ref, sd_ref,
             mxs_ref, mxd_ref):
    hs = jnp.dot(h, w_ref[...], preferred_element_type=F32)
    for q in range(4):
        hs_ref[q, :, :] = hs[:, 16 * q:16 * q + 16]
    ss_t = lax.dot_general(as_ref[...], hs, (((1,), (1,)), ((), ())),
                           preferred_element_type=F32)
    sd_t = lax.dot_general(ad_ref[...], hs, (((1,), (1,)), ((), ())),
                           preferred_element_type=F32)
    ss_ref[...] = ss_t.reshape(1, 1, BLK)
    sd_ref[...] = sd_t.reshape(1, 1, BLK)
    rid = g * BLK + lax.broadcasted_iota(I32, (1, BLK), 1)
    valid = rid < NCELL
    ssm = jnp.max(jnp.where(valid, ss_t, -1e30))
    sdm = jnp.max(jnp.where(valid, sd_t, -1e30))

    @pl.when(g == 0)
    def _():
        mxs_ref[...] = jnp.full((1, 128), -1e30, F32)
        mxd_ref[...] = jnp.full((1, 128), -1e30, F32)

    mxs_ref[...] = jnp.maximum(mxs_ref[...], ssm)
    mxd_ref[...] = jnp.maximum(mxd_ref[...], sdm)


def _embed_outs():
    return (
        jax.ShapeDtypeStruct((4, NPAD, 16), F32),
        jax.ShapeDtypeStruct((NBLK, 1, BLK), F32),
        jax.ShapeDtypeStruct((NBLK, 1, BLK), F32),
        jax.ShapeDtypeStruct((1, 128), F32),
        jax.ShapeDtypeStruct((1, 128), F32),
    )


def _embed_out_specs():
    return (
        pl.BlockSpec((4, BLK, 16), lambda g: (0, g, 0)),
        pl.BlockSpec((1, 1, BLK), lambda g: (g, 0, 0)),
        pl.BlockSpec((1, 1, BLK), lambda g: (g, 0, 0)),
        pl.BlockSpec((1, 128), lambda g: (0, 0)),
        pl.BlockSpec((1, 128), lambda g: (0, 0)),
    )


def _full(shape):
    return pl.BlockSpec(shape, lambda g: tuple(0 for _ in shape))


_tc_h0 = pl.pallas_call(
    _tc_h0_body,
    grid=(NBLK,),
    in_specs=[
        pl.BlockSpec((BLK, 128), lambda g: (g, 0)),
        _full((128, H)), _full((1, H)),
    ],
    out_specs=pl.BlockSpec((4, BLK, 16), lambda g: (0, g, 0)),
    out_shape=jax.ShapeDtypeStruct((4, NPAD, 16), F32),
)

_tc_embed = pl.pallas_call(
    _tc_embed_body,
    grid=(NBLK,),
    in_specs=[
        pl.BlockSpec((4, BLK, 16), lambda g: (0, g, 0)),
        pl.BlockSpec((1, 1, BLK), lambda g: (g, 0, 0)),
        _full((1, 128)),
        _full((1, H)), _full((H, H)), _full((1, H)), _full((1, H)),
    ],
    out_specs=_embed_out_specs(),
    out_shape=_embed_outs(),
)


def _tc_econst_body(attr_ref, wemb_ref, bemb_ref, wes_ref, aes_ref, ec_ref):
    a = attr_ref[...]
    for l in range(3):
        we = wes_ref[pl.ds(64 * l, 64), :]
        v1 = jnp.dot(wemb_ref[...], we, preferred_element_type=F32)
        v0 = jnp.dot(bemb_ref[...], we, preferred_element_type=F32)
        ael = aes_ref[pl.ds(l, 1), :]
        ec_ref[pl.ds(l, 1), :] = jnp.full((1, 128), jnp.sum(v1 * ael), F32)
        ec_ref[pl.ds(3 + l, 1), :] = jnp.full((1, 128), jnp.sum(v0 * ael), F32)
    ec_ref[pl.ds(6, 1), :] = jnp.full((1, 128), jnp.min(a), F32)
    ec_ref[pl.ds(7, 1), :] = jnp.full((1, 128), jnp.max(a), F32)


_tc_econst = pl.pallas_call(
    _tc_econst_body,
    grid=(1,),
    in_specs=[
        _full((6250, 128)),
        _full((1, H)), _full((1, H)), _full((192, H)), _full((3, H)),
    ],
    out_specs=pl.BlockSpec((8, 128), lambda g: (0, 0)),
    out_shape=jax.ShapeDtypeStruct((8, 128), F32),
)


def _tc_wellprep_body(wx_ref, ww_ref, bw_ref, wW_ref, wad_ref,
                      sdw_ref, mxdw_ref):
    hw = jnp.dot(wx_ref[...], ww_ref[...], preferred_element_type=F32)
    hw = hw + bw_ref[...]
    hd = jnp.dot(hw, wW_ref[...], preferred_element_type=F32)
    sd_t = lax.dot_general(wad_ref[...], hd, (((1,), (1,)), ((), ())),
                           preferred_element_type=F32)
    sdw_ref[...] = sd_t
    rid = lax.broadcasted_iota(I32, (1, NWPAD), 1)
    mxdw_ref[...] = jnp.full(
        (1, 128), jnp.max(jnp.where(rid < NWELL, sd_t, -1e30)), F32)


_tc_wellprep = pl.pallas_call(
    _tc_wellprep_body,
    grid=(1,),
    in_specs=[
        _full((NWPAD, 32)), _full((32, H)), _full((1, H)),
        _full((H, H)), _full((1, H)),
    ],
    out_specs=(
        pl.BlockSpec((1, NWPAD), lambda g: (0, 0)),
        pl.BlockSpec((1, 128), lambda g: (0, 0)),
    ),
    out_shape=(
        jax.ShapeDtypeStruct((1, NWPAD), F32),
        jax.ShapeDtypeStruct((1, 128), F32),
    ),
)


def _tc_final_body(outp_ref, denp_ref, wb_ref, m1_ref, mb1_ref, m2_ref,
                   mb2_ref, out_ref):
    hw = jnp.sum(outp_ref[...], axis=0)
    den_row = jnp.sum(denp_ref[...], axis=0).reshape(1, NWPAD)
    ii = lax.broadcasted_iota(I32, (NWPAD, NWPAD), 0)
    jj = lax.broadcasted_iota(I32, (NWPAD, NWPAD), 1)
    ident = jnp.where(ii == jj, 1.0, 0.0).astype(F32)
    den_col = lax.dot_general(ident, den_row, (((1,), (1,)), ((), ())),
                              preferred_element_type=F32)
    hwn = jnp.where(den_col > 0.0, hw / den_col, 0.0) + wb_ref[...]
    z = jnp.dot(hwn, m1_ref[...], preferred_element_type=F32) + mb1_ref[...]
    z = jnp.maximum(z, 0.0)
    out_ref[...] = jnp.dot(z, m2_ref[...], preferred_element_type=F32) + mb2_ref[...]


_tc_final = pl.pallas_call(
    _tc_final_body,
    grid=(1,),
    in_specs=[
        _full((32, NWPAD, 64)), _full((32, NWPAD)), _full((1, H)),
        _full((H, H)), _full((1, H)),
        _full((H, 75)), _full((1, 75)),
    ],
    out_specs=pl.BlockSpec((NWPAD, 75), lambda g: (0, 0)),
    out_shape=jax.ShapeDtypeStruct((NWPAD, 75), F32),
)


_MESH = plsc.VectorSubcoreMesh(core_axis_name="c", subcore_axis_name="s")

_EPT = ECC_PAD // 16
_CH = 128
_NCH = _EPT // _CH
_NHALF = NPAD // 2
_GROW = _NHALF
_ACC = _NHALF
_RPT = _NHALF // 16
_NZ = _RPT // 32


def _make_sc_conv():
    @functools.partial(
        pl.kernel, mesh=_MESH,
        compiler_params=pltpu.CompilerParams(
            use_tc_tiling_on_sc=False, needs_layout_passes=False),
        out_type=(
            jax.ShapeDtypeStruct((4, NPAD, 16), F32),
            jax.ShapeDtypeStruct((2, _NHALF), F32),
        ),
        scratch_types=[
            pltpu.VMEM((2 * NPAD,), F32),
            pltpu.VMEM((3 * _CH,), I32),
            pltpu.VMEM((_CH,), I32),
            pltpu.VMEM((_CH,), I32),
            pltpu.VMEM((_CH,), I32),
            pltpu.VMEM((_CH,), I32),
            pltpu.VMEM((_CH,), F32),
            pltpu.VMEM((_CH,), F32),
            pltpu.VMEM((_CH, 16), F32),
            pltpu.VMEM((_CH, 16), F32),
            pltpu.VMEM((768,), F32),
            pltpu.VMEM_SHARED((_ACC, 16), F32),
            pltpu.VMEM_SHARED((_ACC,), F32),
            pltpu.SemaphoreType.DMA,
            pltpu.SemaphoreType.DMA,
        ],
    )
    def k(epk_r, hs_r, sstab_r, cst_r, out_r, den_r,
          ss_t, ebuf0, idxs0, idxs1, idxd0, idxd1, exs0, exs1,
          rows0, rows1, cbuf, out_sp, den_sp, sem0, sem1):
        c = lax.axis_index("c")
        s = lax.axis_index("s")

        def tl(i, carry):
            pltpu.sync_copy(sstab_r.at[pl.ds(i * 12544, 12544)],
                            ss_t.at[pl.ds(i * 12544, 12544)])
            return carry
        lax.fori_loop(0, 2 * NPAD // 12544, tl, 0)
        pltpu.sync_copy(cst_r, cbuf)
        c1v = cbuf[pl.ds(0, 16)]
        c0v = cbuf[pl.ds(128, 16)]
        aminv = cbuf[pl.ds(256, 16)]
        amaxv = cbuf[pl.ds(384, 16)]
        vmxs = cbuf[pl.ds(512, 16)]
        vmxd = cbuf[pl.ds(640, 16)]
        craw = vmxs + vmxd + jnp.maximum(c1v * aminv, c1v * amaxv) + c0v
        cshift = jnp.maximum(craw, 0.2 * craw)
        zero16 = jnp.zeros((16,), F32)

        def zr(r, carry):
            rows0[r, pl.ds(0, 16)] = zero16
            return carry
        lax.fori_loop(0, _CH, zr, 0)
        for g in range(_CH // 16):
            exs0[pl.ds(16 * g, 16)] = zero16
        zb = s * _RPT
        ebase = s * _EPT
        fb = s * _RPT

        def phase(ph, carry0):
            q = c + 2 * (ph % 2)
            qoff = q * NPAD
            hbase = (ph // 2) * _NHALF
            do_den = (ph % 2) == 0

            def zo(i, carry):
                pltpu.sync_copy(rows0.at[pl.ds(0, 32)],
                                out_sp.at[pl.ds(zb + i * 32, 32)])

                @pl.when(do_den)
                def _():
                    pltpu.sync_copy(exs0.at[pl.ds(0, 32)],
                                    den_sp.at[pl.ds(zb + i * 32, 32)])
                return carry
            lax.fori_loop(0, _NZ, zo, 0)

            plsc.subcore_barrier()

            def start_chunk(ci, eb, ixs, ixd, exb, rb, semx):
                base = ebase + ci * _CH
                pltpu.sync_copy(epk_r.at[pl.ds(base * 3, 3 * _CH)], eb)

                @pl.when(ci > 0)
                def _():
                    pltpu.make_async_copy(rb, out_sp.at[ixd], sem1).wait()
                for g in range(_CH // 16):
                    sv = eb[pl.ds(16 * g, 16)]
                    dv = eb[pl.ds(_CH + 16 * g, 16)]
                    ssv = plsc.load_gather(ss_t, [sv])
                    sdv = plsc.load_gather(ss_t, [dv + NPAD])
                    av = plsc.bitcast(eb[pl.ds(2 * _CH + 16 * g, 16)], F32)
                    raw = ssv + sdv + av * c1v + c0v
                    alpha = jnp.maximum(raw, 0.2 * raw)
                    ex = jnp.exp(alpha - cshift)
                    eid = base + 16 * g + lax.iota(I32, 16)
                    dloc = dv - hbase
                    inh = (dloc >= 0) & (dloc < _NHALF) & (eid < ECC)
                    exb[pl.ds(16 * g, 16)] = jnp.where(inh, ex, 0.0)
                    ixs[pl.ds(16 * g, 16)] = sv + qoff
                    ixd[pl.ds(16 * g, 16)] = jnp.where(inh, dloc, 0)

                pltpu.make_async_copy(hs_r.at[ixs], rb, semx).start()

                @pl.when(do_den)
                def _():
                    pltpu.sync_copy(exb, den_sp.at[ixd], add=True)

            def finish_chunk(ixs, ixd, exb, rb, semx):
                pltpu.make_async_copy(hs_r.at[ixs], rb, semx).wait()

                def rm(r, carry2):
                    iv = jnp.zeros((16,), I32) + r
                    ev = plsc.load_gather(exb, [iv])
                    rb[r, pl.ds(0, 16)] = rb[r, pl.ds(0, 16)] * ev
                    return carry2
                lax.fori_loop(0, _CH, rm, 0)
                pltpu.async_copy(rb, out_sp.at[ixd], sem1, add=True)

            def pipe(j, carry):
                start_chunk(j, ebuf0, idxs0, idxd0, exs0, rows0, sem0)
                finish_chunk(idxs0, idxd0, exs0, rows0, sem0)
                return carry
            lax.fori_loop(0, _NCH, pipe, 0)
            pltpu.make_async_copy(rows0, out_sp.at[idxd0], sem1).wait()
            plsc.subcore_barrier()

            pltpu.sync_copy(out_sp.at[pl.ds(fb, _RPT)],
                            out_r.at[q, pl.ds(hbase + fb, _RPT)])

            @pl.when(do_den & (c == 0))
            def _():
                pltpu.sync_copy(den_sp.at[pl.ds(fb, _RPT)],
                                den_r.at[ph // 2, pl.ds(fb, _RPT)])
            plsc.subcore_barrier()

            def zrr(r, carry):
                rows0[r, pl.ds(0, 16)] = zero16
                return carry
            lax.fori_loop(0, _CH, zrr, 0)
            for g in range(_CH // 16):
                exs0[pl.ds(16 * g, 16)] = zero16
            return carry0

        lax.fori_loop(0, 4, phase, 0)

    return k


_sc_conv = _make_sc_conv()

_EPTW = ECW_PAD // 16
_NCHW = _EPTW // 128
_WRPT = NWPAD // 16


@functools.partial(
    pl.kernel, mesh=_MESH,
    compiler_params=pltpu.CompilerParams(
        use_tc_tiling_on_sc=False, needs_layout_passes=False),
    out_type=(
        jax.ShapeDtypeStruct((32, NWPAD * 64), F32),
        jax.ShapeDtypeStruct((32, NWPAD), F32),
    ),
    scratch_types=[
        pltpu.VMEM((NPAD,), F32),
        pltpu.VMEM((NWPAD,), F32),
        pltpu.VMEM((128,), I32),
        pltpu.VMEM((128,), I32),
        pltpu.VMEM((128,), I32),
        pltpu.VMEM((128,), F32),
        pltpu.VMEM((4, 128, 16), F32),
        pltpu.VMEM((NWPAD * 64,), F32),
        pltpu.VMEM((NWPAD,), F32),
        pltpu.VMEM((16,), F32),
        pltpu.SemaphoreType.DMA,
    ],
)
def _sc_wells(src_r, dst_r, hs_r, ss_r, sd_r, mxs_r, mxd_r, out_r, den_r,
              ss_t, sd_t, idxr, idxq, idxd, exs, rows, acc_t, den_t,
              buf16, sem):
    c = lax.axis_index("c")
    s = lax.axis_index("s")
    w = s * 2 + c
    pltpu.sync_copy(ss_r, ss_t)
    pltpu.sync_copy(sd_r, sd_t)
    pltpu.sync_copy(mxs_r.at[pl.ds(0, 16)], buf16)
    vmxs = buf16[...]
    pltpu.sync_copy(mxd_r.at[pl.ds(0, 16)], buf16)
    vmxd = buf16[...]
    craw = vmxs + vmxd
    cshift = jnp.maximum(craw, 0.2 * craw)
    zero16 = jnp.zeros((16,), F32)

    def zacc(r, carry):
        for q in range(4):
            acc_t[pl.ds(r * 64 + 16 * q, 16)] = zero16
        return carry
    lax.fori_loop(0, NWPAD, zacc, 0)
    for g in range(NWPAD // 16):
        den_t[pl.ds(16 * g, 16)] = zero16
    lane = lax.iota(I32, 16)
    mask0 = lane == 0

    ebase = w * (ECW_PAD // 32)

    def chunk(ci, carry):
        base = ebase + ci * 128
        pltpu.sync_copy(src_r.at[pl.ds(base, 128)], idxr)
        pltpu.sync_copy(dst_r.at[pl.ds(base, 128)], idxd)
        for g in range(8):
            sv = idxr[pl.ds(16 * g, 16)]
            dv = idxd[pl.ds(16 * g, 16)]
            ssv = plsc.load_gather(ss_t, [sv])
            sdv = plsc.load_gather(sd_t, [dv])
            raw = ssv + sdv
            alpha = jnp.maximum(raw, 0.2 * raw)
            ex = jnp.exp(alpha - cshift)
            eid = base + 16 * g + lax.iota(I32, 16)
            ex = jnp.where(eid < ECW, ex, 0.0)
            exs[pl.ds(16 * g, 16)] = ex
        for q in range(4):
            for g in range(8):
                idxq[pl.ds(16 * g, 16)] = (
                    idxr[pl.ds(16 * g, 16)] + q * NPAD)
            pltpu.async_copy(hs_r.at[idxq], rows.at[q], sem).wait()

        def rm(r, carry2):
            iv = jnp.zeros((16,), I32) + r
            ev = plsc.load_gather(exs, [iv])
            d16 = plsc.load_gather(idxd, [iv])
            plsc.addupdate_scatter(den_t, [d16], ev, mask=mask0)
            fbase = d16 * 64 + lane
            for q in range(4):
                rv = rows[q, r, pl.ds(0, 16)] * ev
                plsc.addupdate_scatter(acc_t, [fbase + 16 * q], rv)
            return carry2
        lax.fori_loop(0, 128, rm, 0)
        return carry
    lax.fori_loop(0, ECW_PAD // 32 // 128, chunk, 0)
    pltpu.sync_copy(acc_t, out_r.at[w])
    pltpu.sync_copy(den_t, den_r.at[w])


def kernel(cell_x, well_x, c2c_edge_attr, params, c2c_edge_index,
           c2w_src, c2w_dst):
    p = params
    cx = jnp.zeros((NPAD, 128), F32).at[:NCELL].set(cell_x.astype(F32))
    wx = jnp.zeros((NWPAD, 32), F32).at[:NWELL].set(well_x.astype(F32))
    attr_flat = c2c_edge_attr.astype(F32)[:, 0]
    attr2 = attr_flat.reshape(6250, 128)
    src = c2c_edge_index[0].astype(I32)
    dst = c2c_edge_index[1].astype(I32)
    srcp = jnp.pad(src, (0, ECC_PAD - ECC))
    dstp = jnp.pad(dst, (0, ECC_PAD - ECC))
    attrp = jnp.pad(attr_flat, (0, ECC_PAD - ECC))
    attrb = lax.bitcast_convert_type(attrp, I32)
    epk = jnp.stack(
        [srcp.reshape(-1, 128), dstp.reshape(-1, 128),
         attrb.reshape(-1, 128)], axis=1).reshape(3 * ECC_PAD)
    wsrcp = jnp.pad(c2w_src.astype(I32), (0, ECW_PAD - ECW))
    wdstp = jnp.pad(c2w_dst.astype(I32), (0, ECW_PAD - ECW))

    wemb = p["W_eemb"].astype(F32).reshape(1, H)
    bemb = p["b_eemb"].astype(F32).reshape(1, H)
    wes = jnp.concatenate([c["We"].astype(F32) for c in p["convs"]], axis=0)
    aes = jnp.stack([c["a_e"].astype(F32) for c in p["convs"]], axis=0)
    ec = _tc_econst(attr2, wemb, bemb, wes, aes)
    ec_l = [
        jnp.concatenate(
            [ec[l:l + 1], ec[3 + l:4 + l], ec[6:7], ec[7:8]], axis=0
        ).reshape(512)
        for l in range(3)
    ]

    convs = p["convs"]
    x0 = _tc_h0(cx, p["W_cell"].astype(F32),
                p["b_cell"].astype(F32).reshape(1, H))

    flags = jnp.stack([jnp.full((1, 128), v, F32) for v in (0.0, 1.0, 1.0)])
    bprevs = jnp.stack([jnp.zeros((1, H), F32),
                        convs[0]["b"].astype(F32).reshape(1, H),
                        convs[1]["b"].astype(F32).reshape(1, H)])
    ws = jnp.stack([c["W"].astype(F32) for c in convs])
    ass_ = jnp.stack([c["a_s"].astype(F32).reshape(1, H) for c in convs])
    ads_ = jnp.stack([c["a_d"].astype(F32).reshape(1, H) for c in convs])
    ecs = jnp.stack(ec_l)

    def _layer(i, carry):
        x, den = carry
        flag = lax.dynamic_index_in_dim(flags, i, 0, keepdims=False)
        bprev = lax.dynamic_index_in_dim(bprevs, i, 0, keepdims=False)
        w = lax.dynamic_index_in_dim(ws, i, 0, keepdims=False)
        a_s = lax.dynamic_index_in_dim(ass_, i, 0, keepdims=False)
        a_d = lax.dynamic_index_in_dim(ads_, i, 0, keepdims=False)
        ecl = lax.dynamic_index_in_dim(ecs, i, 0, keepdims=False)
        hs, ss, sd, mxs, mxd = _tc_embed(
            x, den.reshape(NBLK, 1, BLK), flag, bprev, w, a_s, a_d)
        sstab = jnp.concatenate([ss.reshape(NPAD), sd.reshape(NPAD)])
        cst = jnp.concatenate([ecl, mxs.reshape(128), mxd.reshape(128)])
        o, dn = _sc_conv(epk, hs.reshape(4 * NPAD, 16), sstab, cst)
        return (o, dn.reshape(NPAD))

    n_layers = jnp.minimum(srcp[0] * 0 + 3, 3)
    out, den3 = lax.fori_loop(0, n_layers, _layer,
                              (x0, jnp.ones((NPAD,), F32)))

    hsw, ssw, _, mxsw, _ = _tc_embed(
        out, den3.reshape(NBLK, 1, BLK), jnp.full((1, 128), 1.0, F32),
        convs[2]["b"].astype(F32).reshape(1, H),
        p["wW"].astype(F32),
        p["wa_s"].astype(F32).reshape(1, H),
        p["wa_d"].astype(F32).reshape(1, H))

    sdw, mxdw = _tc_wellprep(
        wx, p["W_well"].astype(F32), p["b_well"].astype(F32).reshape(1, H),
        p["wW"].astype(F32), p["wa_d"].astype(F32).reshape(1, H))
    outp, denp = _sc_wells(
        wsrcp, wdstp, hsw.reshape(4 * NPAD, 16), ssw.reshape(NPAD),
        sdw.reshape(NWPAD), mxsw.reshape(128), mxdw.reshape(128))

    out75 = _tc_final(
        outp.reshape(32, NWPAD, 64), denp,
        p["wb"].astype(F32).reshape(1, H),
        p["m1"].astype(F32), p["mb1"].astype(F32).reshape(1, H),
        p["m2"].astype(F32), p["mb2"].astype(F32).reshape(1, 75))
    return out75[:NWELL].reshape(NWELL, 3, 25)
